# baseline (speedup 1.0000x reference)
import functools

import jax
import jax.numpy as jnp
from jax import lax
from jax.experimental import pallas as pl
from jax.experimental.pallas import tpu as pltpu
from jax.experimental.pallas import tpu_sc as plsc

NC = 2
NS = 16
LANES = 16

NU = 25000
NI = 25000
D = 64
E = 800000
LAYERS = 3


def _rsqrt_newton(d):
    i = lax.bitcast_convert_type(d, jnp.int32)
    i = jnp.int32(0x5F3759DF) - lax.shift_right_logical(i, 1)
    y = lax.bitcast_convert_type(i, jnp.float32)
    for _ in range(3):
        y = y * (jnp.float32(1.5) - jnp.float32(0.5) * d * y * y)
    return y


@functools.lru_cache(maxsize=None)
def _build(nu, ni, e, ep_blk, chunk, interpret=False):
    H = NS * 64 * ep_blk
    NSLOT = 2 * H
    assert nu <= H and ni <= H
    RPT = H // NS
    SUP = 3
    NBUF = 3
    align = SUP * chunk
    EPT = -(-e // (NS * align)) * align
    E_PAD = EPT * NS
    NCH = EPT // chunk
    NSUP = NCH // SUP
    CW = 2 * chunk
    SW = SUP * CW
    HPAD = H - nu
    ACCR = H + LANES
    DEGN = -(-(NSLOT + LANES) // (NS * LANES)) * (NS * LANES)
    CAPCH = NCH + 3
    CAPW = CAPCH * CW
    assert RPT % LANES == 0 and chunk % LANES == 0 and NSUP >= 3
    assert EPT % chunk == 0
    NQ = D // LANES
    NG = chunk // LANES

    mesh = plsc.VectorSubcoreMesh(
        core_axis_name="c", subcore_axis_name="s", num_cores=NC,
        num_subcores=NS)
    f32 = jnp.float32
    i32 = jnp.int32
    cparams = pltpu.CompilerParams(
        needs_layout_passes=False, use_tc_tiling_on_sc=False)

    @functools.partial(
        pl.kernel,
        out_type=[jax.ShapeDtypeStruct((NSLOT,), f32),
                  jax.ShapeDtypeStruct((NSLOT, D), f32),
                  jax.ShapeDtypeStruct((NC * NS * CAPW,), i32),
                  jax.ShapeDtypeStruct((NC * NS * LANES,), i32)],
        mesh=mesh,
        scratch_types=[
            pltpu.VMEM((DEGN,), f32),
            pltpu.VMEM((2 * SW,), i32),
            pltpu.VMEM((chunk + 144,), i32),
            pltpu.VMEM((chunk + 144,), i32),
            pltpu.VMEM((CW,), i32),
            pltpu.VMEM((LANES,), i32),
            pltpu.VMEM((RPT,), f32),
            pltpu.VMEM((RPT,), f32),
            pltpu.VMEM((RPT,), f32),
            pltpu.VMEM((64, D), f32),
            pltpu.VMEM((64, D), f32),
            pltpu.VMEM_SHARED((NS * H,), f32),
            pltpu.SemaphoreType.DMA,
            pltpu.SemaphoreType.DMA,
            pltpu.SemaphoreType.DMA,
        ],
        compiler_params=cparams,
        interpret=interpret,
    )
    def k0(pairs_hbm, emb0_hbm, dis_hbm, s_hbm, plist_hbm, cnt_hbm,
           degbuf, stage, prow, pcol, flushbuf, cntbuf, dv, pbuf, disv,
           ebuf, sbuf, degsh, sg0, sg1, fsem):
        c = lax.axis_index("c")
        s = lax.axis_index("s")
        iota = lax.iota(i32, LANES)
        zero16 = jnp.zeros((LANES,), f32)
        one16 = jnp.ones((LANES,), f32)
        tbase = s * NCH * CW
        obase = (c * NS + s) * CAPW
        HB = c * H

        def zdeg(i, _):
            degbuf[pl.ds(i * LANES, LANES)] = zero16
            return 0
        lax.fori_loop(0, DEGN // LANES, zdeg, 0)

        pltpu.async_copy(flushbuf,
                         plist_hbm.at[pl.ds(obase + (CAPCH - 1) * CW, CW)],
                         fsem)

        def flush(p, cnt):
            pltpu.make_async_copy(
                flushbuf, plist_hbm.at[pl.ds(obase, CW)], fsem).wait()
            for i in range(NG):
                sl = pl.ds(i * LANES, LANES)
                flushbuf[sl] = prow[sl]
                flushbuf[pl.ds(chunk + i * LANES, LANES)] = pcol[sl]
            pltpu.async_copy(
                flushbuf, plist_hbm.at[pl.ds(obase + cnt * CW, CW)], fsem)
            for i in range(NG):
                sl = pl.ds(i * LANES, LANES)
                prow[sl] = prow[pl.ds(chunk + i * LANES, LANES)]
                pcol[sl] = pcol[pl.ds(chunk + i * LANES, LANES)]
            return p - chunk

        def do_chunk(u, j, p, cnt):
            sb = (u % 2) * SW + j * CW
            for jj in range(NG):
                cs = stage[pl.ds(sb + chunk + jj * LANES, LANES)]
                slot = cs + jnp.where(cs >= nu, i32(HPAD), i32(0))
                cslot = jnp.where(slot >= NSLOT, NSLOT + iota, slot)
                plsc.addupdate_scatter(degbuf, [cslot], one16)
                r = stage[pl.ds(sb + jj * LANES, LANES)]
                rs = r + jnp.where(r >= nu, i32(HPAD), i32(0))
                loc = slot - HB
                ok = (loc >= 0) & (loc < H)
                plsc.store_compressed(prow.at[pl.ds(p, LANES)], rs, mask=ok)
                plsc.store_compressed(pcol.at[pl.ds(p, LANES)], loc, mask=ok)
                p = p + plsc.all_reduce_population_count(ok)[0]

            @pl.when(p >= chunk)
            def _():
                flush(p, cnt)
            cntn = cnt + jnp.where(p >= chunk, 1, 0)
            pn = jnp.where(p >= chunk, p - chunk, p)
            return pn, cntn

        pltpu.sync_copy(pairs_hbm.at[pl.ds(tbase, SW)], stage.at[pl.ds(0, SW)])
        pltpu.async_copy(pairs_hbm.at[pl.ds(tbase + SW, SW)],
                         stage.at[pl.ds(SW, SW)], sg1)

        def sloop(u, carry):
            p, cnt = carry
            for j in range(SUP):
                p, cnt = do_chunk(u, j, p, cnt)

            @pl.when((u % 2 == 0) & (u < NSUP - 2))
            def _():
                pltpu.async_copy(
                    pairs_hbm.at[pl.ds(tbase + (u + 2) * SW, SW)],
                    stage.at[pl.ds(0, SW)], sg0)

            @pl.when((u % 2 == 1) & (u < NSUP - 2))
            def _():
                pltpu.async_copy(
                    pairs_hbm.at[pl.ds(tbase + (u + 2) * SW, SW)],
                    stage.at[pl.ds(SW, SW)], sg1)
            @pl.when((u + 1) % 2 == 0)
            def _():
                pltpu.make_async_copy(
                    pairs_hbm.at[pl.ds(tbase, SW)],
                    stage.at[pl.ds(0, SW)], sg0).wait()

            @pl.when((u + 1) % 2 == 1)
            def _():
                pltpu.make_async_copy(
                    pairs_hbm.at[pl.ds(tbase, SW)],
                    stage.at[pl.ds(SW, SW)], sg1).wait()
            return p, cnt
        p, cnt = lax.fori_loop(0, NSUP - 1, sloop,
                               (jnp.int32(0), jnp.int32(0)))
        for j in range(SUP):
            p, cnt = do_chunk(NSUP - 1, j, p, cnt)

        @pl.when(p > 0)
        def _():
            for i in range(NG):
                prow[pl.ds(p + i * LANES, LANES)] = jnp.zeros((LANES,), i32)
                pcol[pl.ds(p + i * LANES, LANES)] = H + iota
            flush(jnp.int32(chunk), cnt)
        cnt = jnp.where(p > 0, cnt + 1, cnt)

        def dummy_flush(cnt):
            pltpu.make_async_copy(
                flushbuf, plist_hbm.at[pl.ds(obase, CW)], fsem).wait()
            for i in range(NG):
                flushbuf[pl.ds(i * LANES, LANES)] = jnp.zeros((LANES,), i32)
                flushbuf[pl.ds(chunk + i * LANES, LANES)] = H + iota
            pltpu.async_copy(
                flushbuf, plist_hbm.at[pl.ds(obase + cnt * CW, CW)], fsem)

        for _ in range(SUP - 1):
            @pl.when(cnt % SUP != 0)
            def _():
                dummy_flush(cnt)
            cnt = jnp.where(cnt % SUP != 0, cnt + 1, cnt)
        pltpu.make_async_copy(
            flushbuf, plist_hbm.at[pl.ds(obase, CW)], fsem).wait()
        cntbuf[pl.ds(0, LANES)] = jnp.zeros((LANES,), i32) + cnt
        pltpu.sync_copy(cntbuf, cnt_hbm.at[pl.ds((c * NS + s) * LANES, LANES)])

        pltpu.sync_copy(degbuf.at[pl.ds(c * H, H)], degsh.at[pl.ds(s * H, H)])
        plsc.subcore_barrier()

        gbase = c * H + s * RPT

        def zdv(i, _):
            dv[pl.ds(i * LANES, LANES)] = zero16
            return 0
        lax.fori_loop(0, RPT // LANES, zdv, 0)
        for t in range(NS):
            pltpu.sync_copy(degsh.at[pl.ds(t * H + s * RPT, RPT)], pbuf)

            def acc_part(i, _):
                sl = pl.ds(i * LANES, LANES)
                dv[sl] = dv[sl] + pbuf[sl]
                return 0
            lax.fori_loop(0, RPT // LANES, acc_part, 0)

        def nr(i, _):
            d = dv[pl.ds(i * LANES, LANES)]
            y = _rsqrt_newton(d)
            disv[pl.ds(i * LANES, LANES)] = jnp.where(
                d > jnp.float32(0.5), y, jnp.float32(0.0))
            return 0
        lax.fori_loop(0, RPT // LANES, nr, 0)
        pltpu.sync_copy(disv, dis_hbm.at[pl.ds(gbase, RPT)])

        def blk(b, _):
            r0 = gbase + b * 64
            pltpu.sync_copy(emb0_hbm.at[pl.ds(r0, 64)], ebuf)

            def grp(g, _):
                dvec = disv[pl.ds(b * 64 + g * LANES, LANES)]
                for i in range(LANES):
                    r = g * LANES + i
                    dsc = dvec[i]
                    for q in range(NQ):
                        sl = pl.ds(q * LANES, LANES)
                        sbuf[r, sl] = ebuf[r, sl] * dsc
                return 0
            lax.fori_loop(0, 64 // LANES, grp, 0)
            pltpu.sync_copy(sbuf, s_hbm.at[pl.ds(r0, 64)])
            return 0
        lax.fori_loop(0, ep_blk, blk, 0)

    def make_layer(last):
        outs = [jax.ShapeDtypeStruct((NSLOT, D), f32)]
        if not last:
            outs.append(jax.ShapeDtypeStruct((NSLOT, D), f32))

        @functools.partial(
            pl.kernel,
            out_type=outs,
            mesh=mesh,
            scratch_types=(
                [pltpu.VMEM((chunk,), i32) for _ in range(NBUF)]
                + [pltpu.VMEM((chunk,), i32) for _ in range(NBUF)]
                + [pltpu.VMEM((chunk, D), f32) for _ in range(NBUF)]
                + [
                    pltpu.VMEM((2 * SW,), i32),
                    pltpu.VMEM((LANES,), i32),
                    pltpu.VMEM((RPT,), f32),
                    pltpu.VMEM_SHARED((ACCR, D), f32),
                ]
                + [pltpu.SemaphoreType.DMA for _ in range(2 * NBUF + 2)]
            ),
            compiler_params=cparams,
            interpret=interpret,
        )
        def klayer(plist_hbm, cnt_hbm, s_hbm, dis_hbm, msum_hbm, *rest):
            if last:
                out_hbm, *rest = rest
                sout_hbm = None
            else:
                out_hbm, sout_hbm, *rest = rest
            rowbs = rest[0:NBUF]
            colbs = rest[NBUF:2 * NBUF]
            gbufs = rest[2 * NBUF:3 * NBUF]
            (stage, cntb, disv, accsh) = rest[3 * NBUF:3 * NBUF + 4]
            gsems = rest[3 * NBUF + 4:3 * NBUF + 4 + NBUF]
            ssems = rest[3 * NBUF + 4 + NBUF:3 * NBUF + 4 + 2 * NBUF]
            sg0, sg1 = rest[3 * NBUF + 4 + 2 * NBUF:]
            c = lax.axis_index("c")
            s = lax.axis_index("s")
            zero16 = jnp.zeros((LANES,), f32)
            HB = c * H
            tbase = (c * NS + s) * CAPW

            pltpu.sync_copy(cnt_hbm.at[pl.ds((c * NS + s) * LANES, LANES)],
                            cntb)
            nch = cntb[pl.ds(0, LANES)][0]
            nsup = nch // SUP

            def zrow(i, _):
                for q in range(NQ):
                    gbufs[0][i, pl.ds(q * LANES, LANES)] = zero16
                return 0
            lax.fori_loop(0, 64, zrow, 0)

            def zb(b, _):
                idx = b * NS + s
                pltpu.sync_copy(gbufs[0].at[pl.ds(0, 64)],
                                accsh.at[pl.ds(idx * 64, 64)])
                return 0
            lax.fori_loop(0, H // (64 * NS), zb, 0)
            plsc.subcore_barrier()

            def fire(u, j):
                sb = (u % 2) * SW + j * CW
                for jj in range(NG):
                    sl = pl.ds(jj * LANES, LANES)
                    rowbs[j][sl] = stage[pl.ds(sb + jj * LANES, LANES)]
                    colbs[j][sl] = stage[pl.ds(sb + chunk + jj * LANES,
                                               LANES)]
                pltpu.async_copy(s_hbm.at[rowbs[j]], gbufs[j], gsems[j])

            def scat(j):
                pltpu.make_async_copy(
                    s_hbm.at[rowbs[j]], gbufs[j], gsems[j]).wait()
                pltpu.async_copy(
                    gbufs[j], accsh.at[colbs[j]], ssems[j], add=True)

            def scat_wait(j):
                pltpu.make_async_copy(
                    gbufs[j], accsh.at[colbs[j]], ssems[j]).wait()

            @pl.when(nsup > 0)
            def _():
                pltpu.sync_copy(plist_hbm.at[pl.ds(tbase, SW)],
                                stage.at[pl.ds(0, SW)])

                @pl.when(nsup >= 2)
                def _():
                    pltpu.async_copy(plist_hbm.at[pl.ds(tbase + SW, SW)],
                                     stage.at[pl.ds(SW, SW)], sg1)
                for j in range(SUP):
                    fire(0, j)

                def outer(u, _):
                    @pl.when((u % 2 == 0) & (u < nsup - 2))
                    def _():
                        pltpu.async_copy(
                            plist_hbm.at[pl.ds(tbase + (u + 2) * SW, SW)],
                            stage.at[pl.ds(0, SW)], sg0)

                    @pl.when((u % 2 == 1) & (u < nsup - 2))
                    def _():
                        pltpu.async_copy(
                            plist_hbm.at[pl.ds(tbase + (u + 2) * SW, SW)],
                            stage.at[pl.ds(SW, SW)], sg1)
                    @pl.when((u + 1) % 2 == 0)
                    def _():
                        pltpu.make_async_copy(
                            plist_hbm.at[pl.ds(tbase, SW)],
                            stage.at[pl.ds(0, SW)], sg0).wait()

                    @pl.when((u + 1) % 2 == 1)
                    def _():
                        pltpu.make_async_copy(
                            plist_hbm.at[pl.ds(tbase, SW)],
                            stage.at[pl.ds(SW, SW)], sg1).wait()
                    for j in range(SUP):
                        scat(j)
                    for j in range(SUP):
                        scat_wait(j)
                        fire(u + 1, j)
                    return 0
                lax.fori_loop(0, nsup - 1, outer, 0)
                for j in range(SUP):
                    scat(j)
                for j in range(SUP):
                    scat_wait(j)
            plsc.subcore_barrier()

            lbase = s * RPT
            scale = jnp.float32(1.0 / (LAYERS + 1))
            ga, gb2 = gbufs[0], gbufs[1]
            pltpu.sync_copy(dis_hbm.at[pl.ds(HB + lbase, RPT)], disv)

            def eload(b):
                r0 = lbase + b * 64
                pltpu.async_copy(accsh.at[pl.ds(r0, 64)],
                                 ga.at[pl.ds(0, 64)], gsems[0])
                pltpu.async_copy(msum_hbm.at[pl.ds(HB + r0, 64)],
                                 ga.at[pl.ds(64, 64)], gsems[1])

            def eload_wait(b):
                r0 = lbase + b * 64
                pltpu.make_async_copy(accsh.at[pl.ds(r0, 64)],
                                      ga.at[pl.ds(0, 64)], gsems[0]).wait()
                pltpu.make_async_copy(msum_hbm.at[pl.ds(HB + r0, 64)],
                                      ga.at[pl.ds(64, 64)], gsems[1]).wait()

            def estore(b):
                r0 = lbase + b * 64
                pltpu.async_copy(gb2.at[pl.ds(0, 64)],
                                 out_hbm.at[pl.ds(HB + r0, 64)], ssems[0])
                if not last:
                    pltpu.async_copy(gb2.at[pl.ds(64, 64)],
                                     sout_hbm.at[pl.ds(HB + r0, 64)],
                                     ssems[1])

            def estore_wait(b):
                r0 = lbase + b * 64
                pltpu.make_async_copy(gb2.at[pl.ds(0, 64)],
                                      out_hbm.at[pl.ds(HB + r0, 64)],
                                      ssems[0]).wait()
                if not last:
                    pltpu.make_async_copy(gb2.at[pl.ds(64, 64)],
                                          sout_hbm.at[pl.ds(HB + r0, 64)],
                                          ssems[1]).wait()

            eload(0)

            def blk(b, _):
                eload_wait(b)

                @pl.when(b > 0)
                def _():
                    estore_wait(b - 1)

                def grp(g, _):
                    dvec = disv[pl.ds(b * 64 + g * LANES, LANES)]
                    for i in range(LANES):
                        r = g * LANES + i
                        dsc = dvec[i]
                        for q in range(NQ):
                            sl = pl.ds(q * LANES, LANES)
                            a = ga[r, sl] * dsc
                            if last:
                                gb2[r, sl] = (ga[64 + r, sl] + a) * scale
                            else:
                                gb2[r, sl] = ga[64 + r, sl] + a
                                gb2[64 + r, sl] = a * dsc
                    return 0
                lax.fori_loop(0, 64 // LANES, grp, 0)
                estore(b)

                @pl.when(b + 1 < ep_blk)
                def _():
                    eload(b + 1)
                return 0
            lax.fori_loop(0, ep_blk, blk, 0)
            estore_wait(ep_blk - 1)

        return klayer

    consts = dict(H=H, NSLOT=NSLOT, E_PAD=E_PAD, HPAD=HPAD, NCH=NCH,
                  chunk=chunk)
    return k0, make_layer(False), make_layer(True), consts


def kernel(edge_index, users_emb, items_emb):
    k0, klayer, klayer_last, cc = _build(NU, NI, E, 25, 128)
    H, NSLOT, E_PAD = cc["H"], cc["NSLOT"], cc["E_PAD"]
    NCH, chunk = cc["NCH"], cc["chunk"]
    row = edge_index[0].astype(jnp.int32)
    col = edge_index[1].astype(jnp.int32)
    npad = E_PAD - E
    rowp = jnp.concatenate([row, jnp.zeros((npad,), jnp.int32)])
    colp = jnp.concatenate([col, jnp.full((npad,), NSLOT + 1024, jnp.int32)])
    pairs = jnp.stack([rowp.reshape(NS * NCH, chunk),
                       colp.reshape(NS * NCH, chunk)], axis=1).reshape(-1)
    emb0 = (jnp.zeros((NSLOT, D), jnp.float32)
            .at[:NU].set(users_emb)
            .at[H:H + NI].set(items_emb))
    dis, s0, plist, cnt = k0(pairs, emb0)
    m1, s1 = klayer(plist, cnt, s0, dis, emb0)
    m2, s2 = klayer(plist, cnt, s1, dis, m1)
    fin, = klayer_last(plist, cnt, s2, dis, m2)
    return (fin[:NU], users_emb, fin[H:H + NI], items_emb)


# speedup vs baseline: 1.0799x; 1.0799x over previous
import functools

import jax
import jax.numpy as jnp
from jax import lax
from jax.experimental import pallas as pl
from jax.experimental.pallas import tpu as pltpu
from jax.experimental.pallas import tpu_sc as plsc

NC = 2
NS = 16
LANES = 16

NU = 25000
NI = 25000
D = 64
E = 800000
LAYERS = 3


def _rsqrt_newton(d):
    i = lax.bitcast_convert_type(d, jnp.int32)
    i = jnp.int32(0x5F3759DF) - lax.shift_right_logical(i, 1)
    y = lax.bitcast_convert_type(i, jnp.float32)
    for _ in range(3):
        y = y * (jnp.float32(1.5) - jnp.float32(0.5) * d * y * y)
    return y


@functools.lru_cache(maxsize=None)
def _build(nu, ni, e, ep_blk, chunk, interpret=False):
    H = NS * 64 * ep_blk
    NSLOT = 2 * H
    assert nu <= H and ni <= H
    RPT = H // NS
    SUP = 3
    NBUF = 3
    align = SUP * chunk
    EPT = -(-e // (NS * align)) * align
    E_PAD = EPT * NS
    NCH = EPT // chunk
    NSUP = NCH // SUP
    CW = 2 * chunk
    SW = SUP * CW
    HPAD = H - nu
    ACCR = H + LANES
    DEGN = -(-(NSLOT + LANES) // (NS * LANES)) * (NS * LANES)
    CAPCH = NCH + 3
    CAPW = CAPCH * CW
    assert RPT % LANES == 0 and chunk % LANES == 0 and NSUP >= 3
    assert EPT % chunk == 0
    NQ = D // LANES
    NG = chunk // LANES

    mesh = plsc.VectorSubcoreMesh(
        core_axis_name="c", subcore_axis_name="s", num_cores=NC,
        num_subcores=NS)
    f32 = jnp.float32
    i32 = jnp.int32
    cparams = pltpu.CompilerParams(
        needs_layout_passes=False, use_tc_tiling_on_sc=False)

    @functools.partial(
        pl.kernel,
        out_type=[jax.ShapeDtypeStruct((NSLOT,), f32),
                  jax.ShapeDtypeStruct((NSLOT, D), f32),
                  jax.ShapeDtypeStruct((NC * NS * CAPW,), i32),
                  jax.ShapeDtypeStruct((NC * NS * LANES,), i32)],
        mesh=mesh,
        scratch_types=[
            pltpu.VMEM((DEGN,), f32),
            pltpu.VMEM((2 * SW,), i32),
            pltpu.VMEM((chunk + 144,), i32),
            pltpu.VMEM((chunk + 144,), i32),
            pltpu.VMEM((CW,), i32),
            pltpu.VMEM((LANES,), i32),
            pltpu.VMEM((RPT,), f32),
            pltpu.VMEM((RPT,), f32),
            pltpu.VMEM((RPT,), f32),
            pltpu.VMEM((64, D), f32),
            pltpu.VMEM((64, D), f32),
            pltpu.VMEM_SHARED((NS * H,), f32),
            pltpu.SemaphoreType.DMA,
            pltpu.SemaphoreType.DMA,
            pltpu.SemaphoreType.DMA,
        ],
        compiler_params=cparams,
        interpret=interpret,
    )
    def k0(pairs_hbm, emb0_hbm, dis_hbm, s_hbm, plist_hbm, cnt_hbm,
           degbuf, stage, prow, pcol, flushbuf, cntbuf, dv, pbuf, disv,
           ebuf, sbuf, degsh, sg0, sg1, fsem):
        c = lax.axis_index("c")
        s = lax.axis_index("s")
        iota = lax.iota(i32, LANES)
        zero16 = jnp.zeros((LANES,), f32)
        one16 = jnp.ones((LANES,), f32)
        tbase = s * NCH * CW
        obase = (c * NS + s) * CAPW
        HB = c * H

        def zdeg(i, _):
            degbuf[pl.ds(i * LANES, LANES)] = zero16
            return 0
        lax.fori_loop(0, DEGN // LANES, zdeg, 0)

        pltpu.async_copy(flushbuf,
                         plist_hbm.at[pl.ds(obase + (CAPCH - 1) * CW, CW)],
                         fsem)

        def flush(p, cnt):
            pltpu.make_async_copy(
                flushbuf, plist_hbm.at[pl.ds(obase, CW)], fsem).wait()
            for i in range(NG):
                sl = pl.ds(i * LANES, LANES)
                flushbuf[sl] = prow[sl]
                flushbuf[pl.ds(chunk + i * LANES, LANES)] = pcol[sl]
            pltpu.async_copy(
                flushbuf, plist_hbm.at[pl.ds(obase + cnt * CW, CW)], fsem)
            for i in range(NG):
                sl = pl.ds(i * LANES, LANES)
                prow[sl] = prow[pl.ds(chunk + i * LANES, LANES)]
                pcol[sl] = pcol[pl.ds(chunk + i * LANES, LANES)]
            return p - chunk

        def do_chunk(u, j, p, cnt):
            sb = (u % 2) * SW + j * CW
            for jj in range(NG):
                cs = stage[pl.ds(sb + chunk + jj * LANES, LANES)]
                slot = cs + jnp.where(cs >= nu, i32(HPAD), i32(0))
                cslot = jnp.where(slot >= NSLOT, NSLOT + iota, slot)
                plsc.addupdate_scatter(degbuf, [cslot], one16)
                r = stage[pl.ds(sb + jj * LANES, LANES)]
                rs = r + jnp.where(r >= nu, i32(HPAD), i32(0))
                loc = slot - HB
                ok = (loc >= 0) & (loc < H)
                plsc.store_compressed(prow.at[pl.ds(p, LANES)], rs, mask=ok)
                plsc.store_compressed(pcol.at[pl.ds(p, LANES)], loc, mask=ok)
                p = p + plsc.all_reduce_population_count(ok)[0]

            @pl.when(p >= chunk)
            def _():
                flush(p, cnt)
            cntn = cnt + jnp.where(p >= chunk, 1, 0)
            pn = jnp.where(p >= chunk, p - chunk, p)
            return pn, cntn

        pltpu.sync_copy(pairs_hbm.at[pl.ds(tbase, SW)], stage.at[pl.ds(0, SW)])
        pltpu.async_copy(pairs_hbm.at[pl.ds(tbase + SW, SW)],
                         stage.at[pl.ds(SW, SW)], sg1)

        def sloop(u, carry):
            p, cnt = carry
            for j in range(SUP):
                p, cnt = do_chunk(u, j, p, cnt)

            @pl.when((u % 2 == 0) & (u < NSUP - 2))
            def _():
                pltpu.async_copy(
                    pairs_hbm.at[pl.ds(tbase + (u + 2) * SW, SW)],
                    stage.at[pl.ds(0, SW)], sg0)

            @pl.when((u % 2 == 1) & (u < NSUP - 2))
            def _():
                pltpu.async_copy(
                    pairs_hbm.at[pl.ds(tbase + (u + 2) * SW, SW)],
                    stage.at[pl.ds(SW, SW)], sg1)
            @pl.when((u + 1) % 2 == 0)
            def _():
                pltpu.make_async_copy(
                    pairs_hbm.at[pl.ds(tbase, SW)],
                    stage.at[pl.ds(0, SW)], sg0).wait()

            @pl.when((u + 1) % 2 == 1)
            def _():
                pltpu.make_async_copy(
                    pairs_hbm.at[pl.ds(tbase, SW)],
                    stage.at[pl.ds(SW, SW)], sg1).wait()
            return p, cnt
        p, cnt = lax.fori_loop(0, NSUP - 1, sloop,
                               (jnp.int32(0), jnp.int32(0)))
        for j in range(SUP):
            p, cnt = do_chunk(NSUP - 1, j, p, cnt)

        @pl.when(p > 0)
        def _():
            for i in range(NG):
                prow[pl.ds(p + i * LANES, LANES)] = jnp.zeros((LANES,), i32)
                pcol[pl.ds(p + i * LANES, LANES)] = H + iota
            flush(jnp.int32(chunk), cnt)
        cnt = jnp.where(p > 0, cnt + 1, cnt)

        def dummy_flush(cnt):
            pltpu.make_async_copy(
                flushbuf, plist_hbm.at[pl.ds(obase, CW)], fsem).wait()
            for i in range(NG):
                flushbuf[pl.ds(i * LANES, LANES)] = jnp.zeros((LANES,), i32)
                flushbuf[pl.ds(chunk + i * LANES, LANES)] = H + iota
            pltpu.async_copy(
                flushbuf, plist_hbm.at[pl.ds(obase + cnt * CW, CW)], fsem)

        for _ in range(SUP - 1):
            @pl.when(cnt % SUP != 0)
            def _():
                dummy_flush(cnt)
            cnt = jnp.where(cnt % SUP != 0, cnt + 1, cnt)
        pltpu.make_async_copy(
            flushbuf, plist_hbm.at[pl.ds(obase, CW)], fsem).wait()
        cntbuf[pl.ds(0, LANES)] = jnp.zeros((LANES,), i32) + cnt
        pltpu.sync_copy(cntbuf, cnt_hbm.at[pl.ds((c * NS + s) * LANES, LANES)])

        pltpu.sync_copy(degbuf.at[pl.ds(c * H, H)], degsh.at[pl.ds(s * H, H)])
        plsc.subcore_barrier()

        gbase = c * H + s * RPT

        def zdv(i, _):
            dv[pl.ds(i * LANES, LANES)] = zero16
            return 0
        lax.fori_loop(0, RPT // LANES, zdv, 0)
        for t in range(NS):
            pltpu.sync_copy(degsh.at[pl.ds(t * H + s * RPT, RPT)], pbuf)

            def acc_part(i, _):
                sl = pl.ds(i * LANES, LANES)
                dv[sl] = dv[sl] + pbuf[sl]
                return 0
            lax.fori_loop(0, RPT // LANES, acc_part, 0)

        def nr(i, _):
            d = dv[pl.ds(i * LANES, LANES)]
            y = _rsqrt_newton(d)
            disv[pl.ds(i * LANES, LANES)] = jnp.where(
                d > jnp.float32(0.5), y, jnp.float32(0.0))
            return 0
        lax.fori_loop(0, RPT // LANES, nr, 0)
        pltpu.sync_copy(disv, dis_hbm.at[pl.ds(gbase, RPT)])

        def blk(b, _):
            r0 = gbase + b * 64
            pltpu.sync_copy(emb0_hbm.at[pl.ds(r0, 64)], ebuf)

            def grp(g, _):
                dvec = disv[pl.ds(b * 64 + g * LANES, LANES)]
                for i in range(LANES):
                    r = g * LANES + i
                    dsc = dvec[i]
                    for q in range(NQ):
                        sl = pl.ds(q * LANES, LANES)
                        sbuf[r, sl] = ebuf[r, sl] * dsc
                return 0
            lax.fori_loop(0, 64 // LANES, grp, 0)
            pltpu.sync_copy(sbuf, s_hbm.at[pl.ds(r0, 64)])
            return 0
        lax.fori_loop(0, ep_blk, blk, 0)

    def make_layer(last):
        outs = [jax.ShapeDtypeStruct((NSLOT, D), f32)]
        if not last:
            outs.append(jax.ShapeDtypeStruct((NSLOT, D), f32))

        @functools.partial(
            pl.kernel,
            out_type=outs,
            mesh=mesh,
            scratch_types=(
                [pltpu.VMEM((chunk,), i32) for _ in range(NBUF)]
                + [pltpu.VMEM((chunk,), i32) for _ in range(NBUF)]
                + [pltpu.VMEM((chunk, D), f32) for _ in range(NBUF)]
                + [
                    pltpu.VMEM((2 * SW,), i32),
                    pltpu.VMEM((LANES,), i32),
                    pltpu.VMEM((RPT,), f32),
                    pltpu.VMEM_SHARED((ACCR, D), f32),
                ]
                + [pltpu.SemaphoreType.DMA for _ in range(2 * NBUF + 2)]
            ),
            compiler_params=cparams,
            interpret=interpret,
        )
        def klayer(plist_hbm, cnt_hbm, s_hbm, dis_hbm, msum_hbm, *rest):
            if last:
                out_hbm, *rest = rest
                sout_hbm = None
            else:
                out_hbm, sout_hbm, *rest = rest
            rowbs = rest[0:NBUF]
            colbs = rest[NBUF:2 * NBUF]
            gbufs = rest[2 * NBUF:3 * NBUF]
            (stage, cntb, disv, accsh) = rest[3 * NBUF:3 * NBUF + 4]
            gsems = rest[3 * NBUF + 4:3 * NBUF + 4 + NBUF]
            ssems = rest[3 * NBUF + 4 + NBUF:3 * NBUF + 4 + 2 * NBUF]
            sg0, sg1 = rest[3 * NBUF + 4 + 2 * NBUF:]
            c = lax.axis_index("c")
            s = lax.axis_index("s")
            zero16 = jnp.zeros((LANES,), f32)
            HB = c * H
            tbase = (c * NS + s) * CAPW

            pltpu.sync_copy(cnt_hbm.at[pl.ds((c * NS + s) * LANES, LANES)],
                            cntb)
            nch = cntb[pl.ds(0, LANES)][0]
            nsup = nch // SUP

            def zrow(i, _):
                for q in range(NQ):
                    gbufs[0][i, pl.ds(q * LANES, LANES)] = zero16
                return 0
            lax.fori_loop(0, 64, zrow, 0)

            def zb(b, _):
                idx = b * NS + s
                pltpu.sync_copy(gbufs[0].at[pl.ds(0, 64)],
                                accsh.at[pl.ds(idx * 64, 64)])
                return 0
            lax.fori_loop(0, H // (64 * NS), zb, 0)
            plsc.subcore_barrier()

            def fire(u, j):
                sb = (u % 2) * SW + j * CW
                for jj in range(NG):
                    sl = pl.ds(jj * LANES, LANES)
                    rowbs[j][sl] = stage[pl.ds(sb + jj * LANES, LANES)]
                    colbs[j][sl] = stage[pl.ds(sb + chunk + jj * LANES,
                                               LANES)]
                pltpu.async_copy(s_hbm.at[rowbs[j]], gbufs[j], gsems[j])

            def scat(j):
                pltpu.make_async_copy(
                    s_hbm.at[rowbs[j]], gbufs[j], gsems[j]).wait()
                pltpu.async_copy(
                    gbufs[j], accsh.at[colbs[j]], ssems[j], add=True)

            def scat_wait(j):
                pltpu.make_async_copy(
                    gbufs[j], accsh.at[colbs[j]], ssems[j]).wait()

            @pl.when(nsup > 0)
            def _():
                pltpu.sync_copy(plist_hbm.at[pl.ds(tbase, SW)],
                                stage.at[pl.ds(0, SW)])

                @pl.when(nsup >= 2)
                def _():
                    pltpu.async_copy(plist_hbm.at[pl.ds(tbase + SW, SW)],
                                     stage.at[pl.ds(SW, SW)], sg1)
                for j in range(SUP):
                    fire(0, j)

                def outer(u, _):
                    @pl.when((u % 2 == 0) & (u < nsup - 2))
                    def _():
                        pltpu.async_copy(
                            plist_hbm.at[pl.ds(tbase + (u + 2) * SW, SW)],
                            stage.at[pl.ds(0, SW)], sg0)

                    @pl.when((u % 2 == 1) & (u < nsup - 2))
                    def _():
                        pltpu.async_copy(
                            plist_hbm.at[pl.ds(tbase + (u + 2) * SW, SW)],
                            stage.at[pl.ds(SW, SW)], sg1)
                    @pl.when((u + 1) % 2 == 0)
                    def _():
                        pltpu.make_async_copy(
                            plist_hbm.at[pl.ds(tbase, SW)],
                            stage.at[pl.ds(0, SW)], sg0).wait()

                    @pl.when((u + 1) % 2 == 1)
                    def _():
                        pltpu.make_async_copy(
                            plist_hbm.at[pl.ds(tbase, SW)],
                            stage.at[pl.ds(SW, SW)], sg1).wait()
                    for j in range(SUP):
                        scat(j)
                        scat_wait(j)
                        fire(u + 1, j)
                    return 0
                lax.fori_loop(0, nsup - 1, outer, 0)
                for j in range(SUP):
                    scat(j)
                for j in range(SUP):
                    scat_wait(j)
            plsc.subcore_barrier()

            lbase = s * RPT
            scale = jnp.float32(1.0 / (LAYERS + 1))
            ga, gb2 = gbufs[0], gbufs[1]
            pltpu.sync_copy(dis_hbm.at[pl.ds(HB + lbase, RPT)], disv)

            def eload(b):
                r0 = lbase + b * 64
                pltpu.async_copy(accsh.at[pl.ds(r0, 64)],
                                 ga.at[pl.ds(0, 64)], gsems[0])
                pltpu.async_copy(msum_hbm.at[pl.ds(HB + r0, 64)],
                                 ga.at[pl.ds(64, 64)], gsems[1])

            def eload_wait(b):
                r0 = lbase + b * 64
                pltpu.make_async_copy(accsh.at[pl.ds(r0, 64)],
                                      ga.at[pl.ds(0, 64)], gsems[0]).wait()
                pltpu.make_async_copy(msum_hbm.at[pl.ds(HB + r0, 64)],
                                      ga.at[pl.ds(64, 64)], gsems[1]).wait()

            def estore(b):
                r0 = lbase + b * 64
                pltpu.async_copy(gb2.at[pl.ds(0, 64)],
                                 out_hbm.at[pl.ds(HB + r0, 64)], ssems[0])
                if not last:
                    pltpu.async_copy(gb2.at[pl.ds(64, 64)],
                                     sout_hbm.at[pl.ds(HB + r0, 64)],
                                     ssems[1])

            def estore_wait(b):
                r0 = lbase + b * 64
                pltpu.make_async_copy(gb2.at[pl.ds(0, 64)],
                                      out_hbm.at[pl.ds(HB + r0, 64)],
                                      ssems[0]).wait()
                if not last:
                    pltpu.make_async_copy(gb2.at[pl.ds(64, 64)],
                                          sout_hbm.at[pl.ds(HB + r0, 64)],
                                          ssems[1]).wait()

            eload(0)

            def blk(b, _):
                eload_wait(b)

                @pl.when(b > 0)
                def _():
                    estore_wait(b - 1)

                def grp(g, _):
                    dvec = disv[pl.ds(b * 64 + g * LANES, LANES)]
                    for i in range(LANES):
                        r = g * LANES + i
                        dsc = dvec[i]
                        for q in range(NQ):
                            sl = pl.ds(q * LANES, LANES)
                            a = ga[r, sl] * dsc
                            if last:
                                gb2[r, sl] = (ga[64 + r, sl] + a) * scale
                            else:
                                gb2[r, sl] = ga[64 + r, sl] + a
                                gb2[64 + r, sl] = a * dsc
                    return 0
                lax.fori_loop(0, 64 // LANES, grp, 0)
                estore(b)

                @pl.when(b + 1 < ep_blk)
                def _():
                    eload(b + 1)
                return 0
            lax.fori_loop(0, ep_blk, blk, 0)
            estore_wait(ep_blk - 1)

        return klayer

    consts = dict(H=H, NSLOT=NSLOT, E_PAD=E_PAD, HPAD=HPAD, NCH=NCH,
                  chunk=chunk)
    return k0, make_layer(False), make_layer(True), consts


def kernel(edge_index, users_emb, items_emb):
    k0, klayer, klayer_last, cc = _build(NU, NI, E, 25, 128)
    H, NSLOT, E_PAD = cc["H"], cc["NSLOT"], cc["E_PAD"]
    NCH, chunk = cc["NCH"], cc["chunk"]
    row = edge_index[0].astype(jnp.int32)
    col = edge_index[1].astype(jnp.int32)
    npad = E_PAD - E
    rowp = jnp.concatenate([row, jnp.zeros((npad,), jnp.int32)])
    colp = jnp.concatenate([col, jnp.full((npad,), NSLOT + 1024, jnp.int32)])
    pairs = jnp.stack([rowp.reshape(NS * NCH, chunk),
                       colp.reshape(NS * NCH, chunk)], axis=1).reshape(-1)
    emb0 = (jnp.zeros((NSLOT, D), jnp.float32)
            .at[:NU].set(users_emb)
            .at[H:H + NI].set(items_emb))
    dis, s0, plist, cnt = k0(pairs, emb0)
    m1, s1 = klayer(plist, cnt, s0, dis, emb0)
    m2, s2 = klayer(plist, cnt, s1, dis, m1)
    fin, = klayer_last(plist, cnt, s2, dis, m2)
    return (fin[:NU], users_emb, fin[H:H + NI], items_emb)


# speedup vs baseline: 1.0849x; 1.0047x over previous
import functools

import jax
import jax.numpy as jnp
from jax import lax
from jax.experimental import pallas as pl
from jax.experimental.pallas import tpu as pltpu
from jax.experimental.pallas import tpu_sc as plsc

NC = 2
NS = 16
LANES = 16

NU = 25000
NI = 25000
D = 64
E = 800000
LAYERS = 3


def _rsqrt_newton(d):
    i = lax.bitcast_convert_type(d, jnp.int32)
    i = jnp.int32(0x5F3759DF) - lax.shift_right_logical(i, 1)
    y = lax.bitcast_convert_type(i, jnp.float32)
    for _ in range(3):
        y = y * (jnp.float32(1.5) - jnp.float32(0.5) * d * y * y)
    return y


@functools.lru_cache(maxsize=None)
def _build(nu, ni, e, ep_blk, chunk, interpret=False):
    H = NS * 64 * ep_blk
    NSLOT = 2 * H
    assert nu <= H and ni <= H
    RPT = H // NS
    SUP = 3
    NBUF = 3
    align = SUP * chunk
    EPT = -(-e // (NS * align)) * align
    E_PAD = EPT * NS
    NCH = EPT // chunk
    NSUP = NCH // SUP
    CW = 2 * chunk
    SW = SUP * CW
    HPAD = H - nu
    ACCR = H + LANES
    DEGN = -(-(NSLOT + LANES) // (NS * LANES)) * (NS * LANES)
    CAPCH = NCH + 3
    CAPW = CAPCH * CW
    assert RPT % LANES == 0 and chunk % LANES == 0 and NSUP >= 3
    assert EPT % chunk == 0
    NQ = D // LANES
    NG = chunk // LANES

    mesh = plsc.VectorSubcoreMesh(
        core_axis_name="c", subcore_axis_name="s", num_cores=NC,
        num_subcores=NS)
    f32 = jnp.float32
    i32 = jnp.int32
    cparams = pltpu.CompilerParams(
        needs_layout_passes=False, use_tc_tiling_on_sc=False)

    @functools.partial(
        pl.kernel,
        out_type=[jax.ShapeDtypeStruct((NSLOT,), f32),
                  jax.ShapeDtypeStruct((NSLOT, D), f32),
                  jax.ShapeDtypeStruct((NC * NS * CAPW,), i32),
                  jax.ShapeDtypeStruct((NC * NS * LANES,), i32)],
        mesh=mesh,
        scratch_types=[
            pltpu.VMEM((DEGN,), f32),
            pltpu.VMEM((2 * SW,), i32),
            pltpu.VMEM((chunk + 144,), i32),
            pltpu.VMEM((chunk + 144,), i32),
            pltpu.VMEM((CW,), i32),
            pltpu.VMEM((LANES,), i32),
            pltpu.VMEM((RPT,), f32),
            pltpu.VMEM((RPT,), f32),
            pltpu.VMEM((RPT,), f32),
            pltpu.VMEM((64, D), f32),
            pltpu.VMEM((64, D), f32),
            pltpu.VMEM_SHARED((NS * H,), f32),
            pltpu.SemaphoreType.DMA,
            pltpu.SemaphoreType.DMA,
            pltpu.SemaphoreType.DMA,
        ],
        compiler_params=cparams,
        interpret=interpret,
    )
    def k0(pairs_hbm, emb0_hbm, dis_hbm, s_hbm, plist_hbm, cnt_hbm,
           degbuf, stage, prow, pcol, flushbuf, cntbuf, dv, pbuf, disv,
           ebuf, sbuf, degsh, sg0, sg1, fsem):
        c = lax.axis_index("c")
        s = lax.axis_index("s")
        iota = lax.iota(i32, LANES)
        zero16 = jnp.zeros((LANES,), f32)
        one16 = jnp.ones((LANES,), f32)
        tbase = s * NCH * CW
        obase = (c * NS + s) * CAPW
        HB = c * H

        def zdeg(i, _):
            degbuf[pl.ds(i * LANES, LANES)] = zero16
            return 0
        lax.fori_loop(0, DEGN // LANES, zdeg, 0)

        pltpu.async_copy(flushbuf,
                         plist_hbm.at[pl.ds(obase + (CAPCH - 1) * CW, CW)],
                         fsem)

        def flush(p, cnt):
            pltpu.make_async_copy(
                flushbuf, plist_hbm.at[pl.ds(obase, CW)], fsem).wait()
            for i in range(NG):
                sl = pl.ds(i * LANES, LANES)
                flushbuf[sl] = prow[sl]
                flushbuf[pl.ds(chunk + i * LANES, LANES)] = pcol[sl]
            pltpu.async_copy(
                flushbuf, plist_hbm.at[pl.ds(obase + cnt * CW, CW)], fsem)
            for i in range(NG):
                sl = pl.ds(i * LANES, LANES)
                prow[sl] = prow[pl.ds(chunk + i * LANES, LANES)]
                pcol[sl] = pcol[pl.ds(chunk + i * LANES, LANES)]
            return p - chunk

        def do_chunk(u, j, p, cnt):
            sb = (u % 2) * SW + j * CW
            for jj in range(NG):
                cs = stage[pl.ds(sb + chunk + jj * LANES, LANES)]
                slot = cs + jnp.where(cs >= nu, i32(HPAD), i32(0))
                cslot = jnp.where(slot >= NSLOT, NSLOT + iota, slot)
                plsc.addupdate_scatter(degbuf, [cslot], one16)
                r = stage[pl.ds(sb + jj * LANES, LANES)]
                rs = r + jnp.where(r >= nu, i32(HPAD), i32(0))
                loc = slot - HB
                ok = (loc >= 0) & (loc < H)
                plsc.store_compressed(prow.at[pl.ds(p, LANES)], rs, mask=ok)
                plsc.store_compressed(pcol.at[pl.ds(p, LANES)], loc, mask=ok)
                p = p + plsc.all_reduce_population_count(ok)[0]

            @pl.when(p >= chunk)
            def _():
                flush(p, cnt)
            cntn = cnt + jnp.where(p >= chunk, 1, 0)
            pn = jnp.where(p >= chunk, p - chunk, p)
            return pn, cntn

        pltpu.sync_copy(pairs_hbm.at[pl.ds(tbase, SW)], stage.at[pl.ds(0, SW)])
        pltpu.async_copy(pairs_hbm.at[pl.ds(tbase + SW, SW)],
                         stage.at[pl.ds(SW, SW)], sg1)

        def sloop(u, carry):
            p, cnt = carry
            for j in range(SUP):
                p, cnt = do_chunk(u, j, p, cnt)

            @pl.when((u % 2 == 0) & (u < NSUP - 2))
            def _():
                pltpu.async_copy(
                    pairs_hbm.at[pl.ds(tbase + (u + 2) * SW, SW)],
                    stage.at[pl.ds(0, SW)], sg0)

            @pl.when((u % 2 == 1) & (u < NSUP - 2))
            def _():
                pltpu.async_copy(
                    pairs_hbm.at[pl.ds(tbase + (u + 2) * SW, SW)],
                    stage.at[pl.ds(SW, SW)], sg1)
            @pl.when((u + 1) % 2 == 0)
            def _():
                pltpu.make_async_copy(
                    pairs_hbm.at[pl.ds(tbase, SW)],
                    stage.at[pl.ds(0, SW)], sg0).wait()

            @pl.when((u + 1) % 2 == 1)
            def _():
                pltpu.make_async_copy(
                    pairs_hbm.at[pl.ds(tbase, SW)],
                    stage.at[pl.ds(SW, SW)], sg1).wait()
            return p, cnt
        p, cnt = lax.fori_loop(0, NSUP - 1, sloop,
                               (jnp.int32(0), jnp.int32(0)))
        for j in range(SUP):
            p, cnt = do_chunk(NSUP - 1, j, p, cnt)

        @pl.when(p > 0)
        def _():
            for i in range(NG):
                prow[pl.ds(p + i * LANES, LANES)] = jnp.zeros((LANES,), i32)
                pcol[pl.ds(p + i * LANES, LANES)] = H + iota
            flush(jnp.int32(chunk), cnt)
        cnt = jnp.where(p > 0, cnt + 1, cnt)

        def dummy_flush(cnt):
            pltpu.make_async_copy(
                flushbuf, plist_hbm.at[pl.ds(obase, CW)], fsem).wait()
            for i in range(NG):
                flushbuf[pl.ds(i * LANES, LANES)] = jnp.zeros((LANES,), i32)
                flushbuf[pl.ds(chunk + i * LANES, LANES)] = H + iota
            pltpu.async_copy(
                flushbuf, plist_hbm.at[pl.ds(obase + cnt * CW, CW)], fsem)

        for _ in range(SUP - 1):
            @pl.when(cnt % SUP != 0)
            def _():
                dummy_flush(cnt)
            cnt = jnp.where(cnt % SUP != 0, cnt + 1, cnt)
        pltpu.make_async_copy(
            flushbuf, plist_hbm.at[pl.ds(obase, CW)], fsem).wait()
        cntbuf[pl.ds(0, LANES)] = jnp.zeros((LANES,), i32) + cnt
        pltpu.sync_copy(cntbuf, cnt_hbm.at[pl.ds((c * NS + s) * LANES, LANES)])

        pltpu.sync_copy(degbuf.at[pl.ds(c * H, H)], degsh.at[pl.ds(s * H, H)])
        plsc.subcore_barrier()

        gbase = c * H + s * RPT

        def zdv(i, _):
            dv[pl.ds(i * LANES, LANES)] = zero16
            return 0
        lax.fori_loop(0, RPT // LANES, zdv, 0)
        for t in range(NS):
            pltpu.sync_copy(degsh.at[pl.ds(t * H + s * RPT, RPT)], pbuf)

            def acc_part(i, _):
                sl = pl.ds(i * LANES, LANES)
                dv[sl] = dv[sl] + pbuf[sl]
                return 0
            lax.fori_loop(0, RPT // LANES, acc_part, 0)

        def nr(i, _):
            d = dv[pl.ds(i * LANES, LANES)]
            y = _rsqrt_newton(d)
            disv[pl.ds(i * LANES, LANES)] = jnp.where(
                d > jnp.float32(0.5), y, jnp.float32(0.0))
            return 0
        lax.fori_loop(0, RPT // LANES, nr, 0)
        pltpu.sync_copy(disv, dis_hbm.at[pl.ds(gbase, RPT)])

        def blk(b, _):
            r0 = gbase + b * 64
            pltpu.sync_copy(emb0_hbm.at[pl.ds(r0, 64)], ebuf)

            def grp(g, _):
                dvec = disv[pl.ds(b * 64 + g * LANES, LANES)]
                for i in range(LANES):
                    r = g * LANES + i
                    dsc = dvec[i]
                    for q in range(NQ):
                        sl = pl.ds(q * LANES, LANES)
                        sbuf[r, sl] = ebuf[r, sl] * dsc
                return 0
            lax.fori_loop(0, 64 // LANES, grp, 0)
            pltpu.sync_copy(sbuf, s_hbm.at[pl.ds(r0, 64)])
            return 0
        lax.fori_loop(0, ep_blk, blk, 0)

    def make_layer(last):
        outs = [jax.ShapeDtypeStruct((NSLOT, D), f32)]
        if not last:
            outs.append(jax.ShapeDtypeStruct((NSLOT, D), f32))

        @functools.partial(
            pl.kernel,
            out_type=outs,
            mesh=mesh,
            scratch_types=(
                [pltpu.VMEM((chunk,), i32) for _ in range(NBUF)]
                + [pltpu.VMEM((chunk,), i32) for _ in range(NBUF)]
                + [pltpu.VMEM((chunk, D), f32) for _ in range(NBUF)]
                + [
                    pltpu.VMEM((2 * SW,), i32),
                    pltpu.VMEM((LANES,), i32),
                    pltpu.VMEM((RPT,), f32),
                    pltpu.VMEM_SHARED((ACCR, D), f32),
                ]
                + [pltpu.SemaphoreType.DMA for _ in range(2 * NBUF + 2)]
            ),
            compiler_params=cparams,
            interpret=interpret,
        )
        def klayer(plist_hbm, cnt_hbm, s_hbm, dis_hbm, msum_hbm, *rest):
            if last:
                out_hbm, *rest = rest
                sout_hbm = None
            else:
                out_hbm, sout_hbm, *rest = rest
            rowbs = rest[0:NBUF]
            colbs = rest[NBUF:2 * NBUF]
            gbufs = rest[2 * NBUF:3 * NBUF]
            (stage, cntb, disv, accsh) = rest[3 * NBUF:3 * NBUF + 4]
            gsems = rest[3 * NBUF + 4:3 * NBUF + 4 + NBUF]
            ssems = rest[3 * NBUF + 4 + NBUF:3 * NBUF + 4 + 2 * NBUF]
            sg0, sg1 = rest[3 * NBUF + 4 + 2 * NBUF:]
            c = lax.axis_index("c")
            s = lax.axis_index("s")
            zero16 = jnp.zeros((LANES,), f32)
            HB = c * H
            tbase = (c * NS + s) * CAPW

            pltpu.sync_copy(cnt_hbm.at[pl.ds((c * NS + s) * LANES, LANES)],
                            cntb)
            nch = cntb[pl.ds(0, LANES)][0]
            nsup = nch // SUP

            def zrow(i, _):
                for q in range(NQ):
                    gbufs[0][i, pl.ds(q * LANES, LANES)] = zero16
                return 0
            lax.fori_loop(0, 64, zrow, 0)

            def zb(b, _):
                idx = b * NS + s
                pltpu.async_copy(gbufs[0].at[pl.ds(0, 64)],
                                 accsh.at[pl.ds(idx * 64, 64)], ssems[0])
                return 0
            lax.fori_loop(0, H // (64 * NS), zb, 0)

            def zbw(b, _):
                idx = b * NS + s
                pltpu.make_async_copy(gbufs[0].at[pl.ds(0, 64)],
                                      accsh.at[pl.ds(idx * 64, 64)],
                                      ssems[0]).wait()
                return 0
            lax.fori_loop(0, H // (64 * NS), zbw, 0)
            plsc.subcore_barrier()

            def fire(u, j):
                sb = (u % 2) * SW + j * CW
                for jj in range(NG):
                    sl = pl.ds(jj * LANES, LANES)
                    rowbs[j][sl] = stage[pl.ds(sb + jj * LANES, LANES)]
                    colbs[j][sl] = stage[pl.ds(sb + chunk + jj * LANES,
                                               LANES)]
                pltpu.async_copy(s_hbm.at[rowbs[j]], gbufs[j], gsems[j])

            def scat(j):
                pltpu.make_async_copy(
                    s_hbm.at[rowbs[j]], gbufs[j], gsems[j]).wait()
                pltpu.async_copy(
                    gbufs[j], accsh.at[colbs[j]], ssems[j], add=True)

            def scat_wait(j):
                pltpu.make_async_copy(
                    gbufs[j], accsh.at[colbs[j]], ssems[j]).wait()

            @pl.when(nsup > 0)
            def _():
                pltpu.sync_copy(plist_hbm.at[pl.ds(tbase, SW)],
                                stage.at[pl.ds(0, SW)])

                @pl.when(nsup >= 2)
                def _():
                    pltpu.async_copy(plist_hbm.at[pl.ds(tbase + SW, SW)],
                                     stage.at[pl.ds(SW, SW)], sg1)
                for j in range(SUP):
                    fire(0, j)

                def outer(u, _):
                    @pl.when((u % 2 == 0) & (u < nsup - 2))
                    def _():
                        pltpu.async_copy(
                            plist_hbm.at[pl.ds(tbase + (u + 2) * SW, SW)],
                            stage.at[pl.ds(0, SW)], sg0)

                    @pl.when((u % 2 == 1) & (u < nsup - 2))
                    def _():
                        pltpu.async_copy(
                            plist_hbm.at[pl.ds(tbase + (u + 2) * SW, SW)],
                            stage.at[pl.ds(SW, SW)], sg1)
                    @pl.when((u + 1) % 2 == 0)
                    def _():
                        pltpu.make_async_copy(
                            plist_hbm.at[pl.ds(tbase, SW)],
                            stage.at[pl.ds(0, SW)], sg0).wait()

                    @pl.when((u + 1) % 2 == 1)
                    def _():
                        pltpu.make_async_copy(
                            plist_hbm.at[pl.ds(tbase, SW)],
                            stage.at[pl.ds(SW, SW)], sg1).wait()
                    for j in range(SUP):
                        scat(j)
                        scat_wait(j)
                        fire(u + 1, j)
                    return 0
                lax.fori_loop(0, nsup - 1, outer, 0)
                for j in range(SUP):
                    scat(j)
                for j in range(SUP):
                    scat_wait(j)
            plsc.subcore_barrier()

            lbase = s * RPT
            scale = jnp.float32(1.0 / (LAYERS + 1))
            ga, gb2 = gbufs[0], gbufs[1]
            pltpu.sync_copy(dis_hbm.at[pl.ds(HB + lbase, RPT)], disv)

            def eload(b):
                r0 = lbase + b * 64
                pltpu.async_copy(accsh.at[pl.ds(r0, 64)],
                                 ga.at[pl.ds(0, 64)], gsems[0])
                pltpu.async_copy(msum_hbm.at[pl.ds(HB + r0, 64)],
                                 ga.at[pl.ds(64, 64)], gsems[1])

            def eload_wait(b):
                r0 = lbase + b * 64
                pltpu.make_async_copy(accsh.at[pl.ds(r0, 64)],
                                      ga.at[pl.ds(0, 64)], gsems[0]).wait()
                pltpu.make_async_copy(msum_hbm.at[pl.ds(HB + r0, 64)],
                                      ga.at[pl.ds(64, 64)], gsems[1]).wait()

            def estore(b):
                r0 = lbase + b * 64
                pltpu.async_copy(gb2.at[pl.ds(0, 64)],
                                 out_hbm.at[pl.ds(HB + r0, 64)], ssems[0])
                if not last:
                    pltpu.async_copy(gb2.at[pl.ds(64, 64)],
                                     sout_hbm.at[pl.ds(HB + r0, 64)],
                                     ssems[1])

            def estore_wait(b):
                r0 = lbase + b * 64
                pltpu.make_async_copy(gb2.at[pl.ds(0, 64)],
                                      out_hbm.at[pl.ds(HB + r0, 64)],
                                      ssems[0]).wait()
                if not last:
                    pltpu.make_async_copy(gb2.at[pl.ds(64, 64)],
                                          sout_hbm.at[pl.ds(HB + r0, 64)],
                                          ssems[1]).wait()

            eload(0)

            def blk(b, _):
                eload_wait(b)

                @pl.when(b > 0)
                def _():
                    estore_wait(b - 1)

                def grp(g, _):
                    dvec = disv[pl.ds(b * 64 + g * LANES, LANES)]
                    for i in range(LANES):
                        r = g * LANES + i
                        dsc = dvec[i]
                        for q in range(NQ):
                            sl = pl.ds(q * LANES, LANES)
                            a = ga[r, sl] * dsc
                            if last:
                                gb2[r, sl] = (ga[64 + r, sl] + a) * scale
                            else:
                                gb2[r, sl] = ga[64 + r, sl] + a
                                gb2[64 + r, sl] = a * dsc
                    return 0
                lax.fori_loop(0, 64 // LANES, grp, 0)
                estore(b)

                @pl.when(b + 1 < ep_blk)
                def _():
                    eload(b + 1)
                return 0
            lax.fori_loop(0, ep_blk, blk, 0)
            estore_wait(ep_blk - 1)

        return klayer

    consts = dict(H=H, NSLOT=NSLOT, E_PAD=E_PAD, HPAD=HPAD, NCH=NCH,
                  chunk=chunk)
    return k0, make_layer(False), make_layer(True), consts


def kernel(edge_index, users_emb, items_emb):
    k0, klayer, klayer_last, cc = _build(NU, NI, E, 25, 128)
    H, NSLOT, E_PAD = cc["H"], cc["NSLOT"], cc["E_PAD"]
    NCH, chunk = cc["NCH"], cc["chunk"]
    row = edge_index[0].astype(jnp.int32)
    col = edge_index[1].astype(jnp.int32)
    npad = E_PAD - E
    rowp = jnp.concatenate([row, jnp.zeros((npad,), jnp.int32)])
    colp = jnp.concatenate([col, jnp.full((npad,), NSLOT + 1024, jnp.int32)])
    pairs = jnp.stack([rowp.reshape(NS * NCH, chunk),
                       colp.reshape(NS * NCH, chunk)], axis=1).reshape(-1)
    emb0 = (jnp.zeros((NSLOT, D), jnp.float32)
            .at[:NU].set(users_emb)
            .at[H:H + NI].set(items_emb))
    dis, s0, plist, cnt = k0(pairs, emb0)
    m1, s1 = klayer(plist, cnt, s0, dis, emb0)
    m2, s2 = klayer(plist, cnt, s1, dis, m1)
    fin, = klayer_last(plist, cnt, s2, dis, m2)
    return (fin[:NU], users_emb, fin[H:H + NI], items_emb)


# speedup vs baseline: 1.1084x; 1.0217x over previous
import functools

import jax
import jax.numpy as jnp
from jax import lax
from jax.experimental import pallas as pl
from jax.experimental.pallas import tpu as pltpu
from jax.experimental.pallas import tpu_sc as plsc

NC = 2
NS = 16
LANES = 16

NU = 25000
NI = 25000
D = 64
E = 800000
LAYERS = 3


def _rsqrt_newton(d):
    i = lax.bitcast_convert_type(d, jnp.int32)
    i = jnp.int32(0x5F3759DF) - lax.shift_right_logical(i, 1)
    y = lax.bitcast_convert_type(i, jnp.float32)
    for _ in range(3):
        y = y * (jnp.float32(1.5) - jnp.float32(0.5) * d * y * y)
    return y


@functools.lru_cache(maxsize=None)
def _build(nu, ni, e, ep_blk, chunk, interpret=False):
    H = NS * 64 * ep_blk
    NSLOT = 2 * H
    assert nu <= H and ni <= H
    RPT = H // NS
    SUP = 3
    NBUF = 3
    align = SUP * chunk
    EPT = -(-e // (NS * align)) * align
    E_PAD = EPT * NS
    NCH = EPT // chunk
    NSUP = NCH // SUP
    CW = 2 * chunk
    SW = SUP * CW
    HPAD = H - nu
    ACCR = H + LANES
    DEGN = -(-(NSLOT + LANES) // (NS * LANES)) * (NS * LANES)
    CAPCH = NCH + 3
    CAPW = CAPCH * CW
    assert RPT % LANES == 0 and chunk % LANES == 0 and NSUP >= 3
    assert EPT % chunk == 0
    NQ = D // LANES
    NG = chunk // LANES

    mesh = plsc.VectorSubcoreMesh(
        core_axis_name="c", subcore_axis_name="s", num_cores=NC,
        num_subcores=NS)
    f32 = jnp.float32
    i32 = jnp.int32
    cparams = pltpu.CompilerParams(
        needs_layout_passes=False, use_tc_tiling_on_sc=False)

    @functools.partial(
        pl.kernel,
        out_type=[jax.ShapeDtypeStruct((NSLOT,), f32),
                  jax.ShapeDtypeStruct((NSLOT, D), f32),
                  jax.ShapeDtypeStruct((NC * NS * CAPW,), i32),
                  jax.ShapeDtypeStruct((NC * NS * LANES,), i32)],
        mesh=mesh,
        scratch_types=[
            pltpu.VMEM((DEGN,), f32),
            pltpu.VMEM((2 * SW,), i32),
            pltpu.VMEM((chunk + 144,), i32),
            pltpu.VMEM((chunk + 144,), i32),
            pltpu.VMEM((CW,), i32),
            pltpu.VMEM((LANES,), i32),
            pltpu.VMEM((RPT,), f32),
            pltpu.VMEM((RPT,), f32),
            pltpu.VMEM((RPT,), f32),
            pltpu.VMEM((64, D), f32),
            pltpu.VMEM((64, D), f32),
            pltpu.VMEM_SHARED((NS * H,), f32),
            pltpu.SemaphoreType.DMA,
            pltpu.SemaphoreType.DMA,
            pltpu.SemaphoreType.DMA,
        ],
        compiler_params=cparams,
        interpret=interpret,
    )
    def k0(pairs_hbm, emb0_hbm, dis_hbm, s_hbm, plist_hbm, cnt_hbm,
           degbuf, stage, prow, pcol, flushbuf, cntbuf, dv, pbuf, disv,
           ebuf, sbuf, degsh, sg0, sg1, fsem):
        c = lax.axis_index("c")
        s = lax.axis_index("s")
        iota = lax.iota(i32, LANES)
        zero16 = jnp.zeros((LANES,), f32)
        one16 = jnp.ones((LANES,), f32)
        tbase = s * NCH * CW
        obase = (c * NS + s) * CAPW
        HB = c * H

        def zdeg(i, _):
            degbuf[pl.ds(i * LANES, LANES)] = zero16
            return 0
        lax.fori_loop(0, DEGN // LANES, zdeg, 0)

        pltpu.async_copy(flushbuf,
                         plist_hbm.at[pl.ds(obase + (CAPCH - 1) * CW, CW)],
                         fsem)

        def flush(p, cnt):
            pltpu.make_async_copy(
                flushbuf, plist_hbm.at[pl.ds(obase, CW)], fsem).wait()
            for i in range(NG):
                sl = pl.ds(i * LANES, LANES)
                flushbuf[sl] = prow[sl]
                flushbuf[pl.ds(chunk + i * LANES, LANES)] = pcol[sl]
            pltpu.async_copy(
                flushbuf, plist_hbm.at[pl.ds(obase + cnt * CW, CW)], fsem)
            for i in range(NG):
                sl = pl.ds(i * LANES, LANES)
                prow[sl] = prow[pl.ds(chunk + i * LANES, LANES)]
                pcol[sl] = pcol[pl.ds(chunk + i * LANES, LANES)]
            return p - chunk

        def do_chunk(u, j, p, cnt):
            sb = (u % 2) * SW + j * CW
            for jj in range(NG):
                cs = stage[pl.ds(sb + chunk + jj * LANES, LANES)]
                slot = cs + jnp.where(cs >= nu, i32(HPAD), i32(0))
                cslot = jnp.where(slot >= NSLOT, NSLOT + iota, slot)
                plsc.addupdate_scatter(degbuf, [cslot], one16)
                r = stage[pl.ds(sb + jj * LANES, LANES)]
                rs = r + jnp.where(r >= nu, i32(HPAD), i32(0))
                loc = slot - HB
                ok = (loc >= 0) & (loc < H)
                plsc.store_compressed(prow.at[pl.ds(p, LANES)], rs, mask=ok)
                plsc.store_compressed(pcol.at[pl.ds(p, LANES)], loc, mask=ok)
                p = p + plsc.all_reduce_population_count(ok)[0]

            @pl.when(p >= chunk)
            def _():
                flush(p, cnt)
            cntn = cnt + jnp.where(p >= chunk, 1, 0)
            pn = jnp.where(p >= chunk, p - chunk, p)
            return pn, cntn

        pltpu.sync_copy(pairs_hbm.at[pl.ds(tbase, SW)], stage.at[pl.ds(0, SW)])
        pltpu.async_copy(pairs_hbm.at[pl.ds(tbase + SW, SW)],
                         stage.at[pl.ds(SW, SW)], sg1)

        def sloop(u, carry):
            p, cnt = carry
            for j in range(SUP):
                p, cnt = do_chunk(u, j, p, cnt)

            @pl.when((u % 2 == 0) & (u < NSUP - 2))
            def _():
                pltpu.async_copy(
                    pairs_hbm.at[pl.ds(tbase + (u + 2) * SW, SW)],
                    stage.at[pl.ds(0, SW)], sg0)

            @pl.when((u % 2 == 1) & (u < NSUP - 2))
            def _():
                pltpu.async_copy(
                    pairs_hbm.at[pl.ds(tbase + (u + 2) * SW, SW)],
                    stage.at[pl.ds(SW, SW)], sg1)
            @pl.when((u + 1) % 2 == 0)
            def _():
                pltpu.make_async_copy(
                    pairs_hbm.at[pl.ds(tbase, SW)],
                    stage.at[pl.ds(0, SW)], sg0).wait()

            @pl.when((u + 1) % 2 == 1)
            def _():
                pltpu.make_async_copy(
                    pairs_hbm.at[pl.ds(tbase, SW)],
                    stage.at[pl.ds(SW, SW)], sg1).wait()
            return p, cnt
        p, cnt = lax.fori_loop(0, NSUP - 1, sloop,
                               (jnp.int32(0), jnp.int32(0)))
        for j in range(SUP):
            p, cnt = do_chunk(NSUP - 1, j, p, cnt)

        @pl.when(p > 0)
        def _():
            for i in range(NG):
                prow[pl.ds(p + i * LANES, LANES)] = jnp.zeros((LANES,), i32)
                pcol[pl.ds(p + i * LANES, LANES)] = H + iota
            flush(jnp.int32(chunk), cnt)
        cnt = jnp.where(p > 0, cnt + 1, cnt)

        def dummy_flush(cnt):
            pltpu.make_async_copy(
                flushbuf, plist_hbm.at[pl.ds(obase, CW)], fsem).wait()
            for i in range(NG):
                flushbuf[pl.ds(i * LANES, LANES)] = jnp.zeros((LANES,), i32)
                flushbuf[pl.ds(chunk + i * LANES, LANES)] = H + iota
            pltpu.async_copy(
                flushbuf, plist_hbm.at[pl.ds(obase + cnt * CW, CW)], fsem)

        for _ in range(SUP - 1):
            @pl.when(cnt % SUP != 0)
            def _():
                dummy_flush(cnt)
            cnt = jnp.where(cnt % SUP != 0, cnt + 1, cnt)
        pltpu.make_async_copy(
            flushbuf, plist_hbm.at[pl.ds(obase, CW)], fsem).wait()
        cntbuf[pl.ds(0, LANES)] = jnp.zeros((LANES,), i32) + cnt
        pltpu.sync_copy(cntbuf, cnt_hbm.at[pl.ds((c * NS + s) * LANES, LANES)])

        pltpu.sync_copy(degbuf.at[pl.ds(c * H, H)], degsh.at[pl.ds(s * H, H)])
        plsc.subcore_barrier()

        gbase = c * H + s * RPT

        def zdv(i, _):
            dv[pl.ds(i * LANES, LANES)] = zero16
            return 0
        lax.fori_loop(0, RPT // LANES, zdv, 0)
        for t in range(NS):
            pltpu.sync_copy(degsh.at[pl.ds(t * H + s * RPT, RPT)], pbuf)

            def acc_part(i, _):
                sl = pl.ds(i * LANES, LANES)
                dv[sl] = dv[sl] + pbuf[sl]
                return 0
            lax.fori_loop(0, RPT // LANES, acc_part, 0)

        def nr(i, _):
            d = dv[pl.ds(i * LANES, LANES)]
            y = _rsqrt_newton(d)
            disv[pl.ds(i * LANES, LANES)] = jnp.where(
                d > jnp.float32(0.5), y, jnp.float32(0.0))
            return 0
        lax.fori_loop(0, RPT // LANES, nr, 0)
        pltpu.sync_copy(disv, dis_hbm.at[pl.ds(gbase, RPT)])

        def blk(b, _):
            r0 = gbase + b * 64
            pltpu.sync_copy(emb0_hbm.at[pl.ds(r0, 64)], ebuf)

            def grp(g, _):
                dvec = disv[pl.ds(b * 64 + g * LANES, LANES)]
                for i in range(LANES):
                    r = g * LANES + i
                    dsc = dvec[i]
                    for q in range(NQ):
                        sl = pl.ds(q * LANES, LANES)
                        sbuf[r, sl] = ebuf[r, sl] * dsc
                return 0
            lax.fori_loop(0, 64 // LANES, grp, 0)
            pltpu.sync_copy(sbuf, s_hbm.at[pl.ds(r0, 64)])
            return 0
        lax.fori_loop(0, ep_blk, blk, 0)

    def make_layer(last):
        if last:
            outs = [jax.ShapeDtypeStruct((nu, D), f32),
                    jax.ShapeDtypeStruct((ni, D), f32)]
        else:
            outs = [jax.ShapeDtypeStruct((NSLOT, D), f32),
                    jax.ShapeDtypeStruct((NSLOT, D), f32)]

        @functools.partial(
            pl.kernel,
            out_type=outs,
            mesh=mesh,
            scratch_types=(
                [pltpu.VMEM((chunk,), i32) for _ in range(NBUF)]
                + [pltpu.VMEM((chunk,), i32) for _ in range(NBUF)]
                + [pltpu.VMEM((chunk, D), f32) for _ in range(NBUF)]
                + [
                    pltpu.VMEM((2 * SW,), i32),
                    pltpu.VMEM((LANES,), i32),
                    pltpu.VMEM((RPT,), f32),
                    pltpu.VMEM_SHARED((ACCR, D), f32),
                ]
                + [pltpu.SemaphoreType.DMA for _ in range(2 * NBUF + 2)]
            ),
            compiler_params=cparams,
            interpret=interpret,
        )
        def klayer(plist_hbm, cnt_hbm, s_hbm, dis_hbm, msum_hbm, *rest):
            if last:
                uout_hbm, iout_hbm, *rest = rest
                out_hbm = sout_hbm = None
            else:
                out_hbm, sout_hbm, *rest = rest
                uout_hbm = iout_hbm = None
            rowbs = rest[0:NBUF]
            colbs = rest[NBUF:2 * NBUF]
            gbufs = rest[2 * NBUF:3 * NBUF]
            (stage, cntb, disv, accsh) = rest[3 * NBUF:3 * NBUF + 4]
            gsems = rest[3 * NBUF + 4:3 * NBUF + 4 + NBUF]
            ssems = rest[3 * NBUF + 4 + NBUF:3 * NBUF + 4 + 2 * NBUF]
            sg0, sg1 = rest[3 * NBUF + 4 + 2 * NBUF:]
            c = lax.axis_index("c")
            s = lax.axis_index("s")
            zero16 = jnp.zeros((LANES,), f32)
            HB = c * H
            tbase = (c * NS + s) * CAPW

            pltpu.sync_copy(cnt_hbm.at[pl.ds((c * NS + s) * LANES, LANES)],
                            cntb)
            nch = cntb[pl.ds(0, LANES)][0]
            nsup = nch // SUP

            def zrow(i, _):
                for q in range(NQ):
                    gbufs[0][i, pl.ds(q * LANES, LANES)] = zero16
                return 0
            lax.fori_loop(0, 64, zrow, 0)

            def zb(b, _):
                idx = b * NS + s
                pltpu.sync_copy(gbufs[0].at[pl.ds(0, 64)],
                                accsh.at[pl.ds(idx * 64, 64)])
                return 0
            lax.fori_loop(0, H // (64 * NS), zb, 0)
            plsc.subcore_barrier()

            def fire(u, j):
                sb = (u % 2) * SW + j * CW
                for jj in range(NG):
                    sl = pl.ds(jj * LANES, LANES)
                    rowbs[j][sl] = stage[pl.ds(sb + jj * LANES, LANES)]
                    colbs[j][sl] = stage[pl.ds(sb + chunk + jj * LANES,
                                               LANES)]
                pltpu.async_copy(s_hbm.at[rowbs[j]], gbufs[j], gsems[j])

            def scat(j):
                pltpu.make_async_copy(
                    s_hbm.at[rowbs[j]], gbufs[j], gsems[j]).wait()
                pltpu.async_copy(
                    gbufs[j], accsh.at[colbs[j]], ssems[j], add=True)

            def scat_wait(j):
                pltpu.make_async_copy(
                    gbufs[j], accsh.at[colbs[j]], ssems[j]).wait()

            @pl.when(nsup > 0)
            def _():
                pltpu.sync_copy(plist_hbm.at[pl.ds(tbase, SW)],
                                stage.at[pl.ds(0, SW)])

                @pl.when(nsup >= 2)
                def _():
                    pltpu.async_copy(plist_hbm.at[pl.ds(tbase + SW, SW)],
                                     stage.at[pl.ds(SW, SW)], sg1)
                for j in range(SUP):
                    fire(0, j)

                def outer(u, _):
                    @pl.when((u % 2 == 0) & (u < nsup - 2))
                    def _():
                        pltpu.async_copy(
                            plist_hbm.at[pl.ds(tbase + (u + 2) * SW, SW)],
                            stage.at[pl.ds(0, SW)], sg0)

                    @pl.when((u % 2 == 1) & (u < nsup - 2))
                    def _():
                        pltpu.async_copy(
                            plist_hbm.at[pl.ds(tbase + (u + 2) * SW, SW)],
                            stage.at[pl.ds(SW, SW)], sg1)
                    @pl.when((u + 1) % 2 == 0)
                    def _():
                        pltpu.make_async_copy(
                            plist_hbm.at[pl.ds(tbase, SW)],
                            stage.at[pl.ds(0, SW)], sg0).wait()

                    @pl.when((u + 1) % 2 == 1)
                    def _():
                        pltpu.make_async_copy(
                            plist_hbm.at[pl.ds(tbase, SW)],
                            stage.at[pl.ds(SW, SW)], sg1).wait()
                    for j in range(SUP):
                        scat(j)
                        scat_wait(j)
                        fire(u + 1, j)
                    return 0
                lax.fori_loop(0, nsup - 1, outer, 0)
                for j in range(SUP):
                    scat(j)
                for j in range(SUP):
                    scat_wait(j)
            plsc.subcore_barrier()

            lbase = s * RPT
            scale = jnp.float32(1.0 / (LAYERS + 1))
            ga, gb2 = gbufs[0], gbufs[1]
            pltpu.sync_copy(dis_hbm.at[pl.ds(HB + lbase, RPT)], disv)

            def eload(b):
                r0 = lbase + b * 64
                pltpu.async_copy(accsh.at[pl.ds(r0, 64)],
                                 ga.at[pl.ds(0, 64)], gsems[0])
                pltpu.async_copy(msum_hbm.at[pl.ds(HB + r0, 64)],
                                 ga.at[pl.ds(64, 64)], gsems[1])

            def eload_wait(b):
                r0 = lbase + b * 64
                pltpu.make_async_copy(accsh.at[pl.ds(r0, 64)],
                                      ga.at[pl.ds(0, 64)], gsems[0]).wait()
                pltpu.make_async_copy(msum_hbm.at[pl.ds(HB + r0, 64)],
                                      ga.at[pl.ds(64, 64)], gsems[1]).wait()

            NR = nu % 64

            def last_out_dma(r0, issue):
                for half, dst in ((0, uout_hbm), (1, iout_hbm)):
                    nreal = nu if half == 0 else ni

                    @pl.when((c == half) & (r0 + 64 <= nreal))
                    def _():
                        cp = pltpu.make_async_copy(
                            gb2.at[pl.ds(0, 64)], dst.at[pl.ds(r0, 64)],
                            ssems[0])
                        cp.start() if issue else cp.wait()
                    if NR > 0:
                        @pl.when((c == half) & (r0 < nreal)
                                 & (r0 + 64 > nreal))
                        def _():
                            cp = pltpu.make_async_copy(
                                gb2.at[pl.ds(0, NR)],
                                dst.at[pl.ds(r0, NR)], ssems[0])
                            cp.start() if issue else cp.wait()

            def estore(b):
                r0 = lbase + b * 64
                if last:
                    last_out_dma(r0, True)
                else:
                    pltpu.async_copy(gb2.at[pl.ds(0, 64)],
                                     out_hbm.at[pl.ds(HB + r0, 64)], ssems[0])
                    pltpu.async_copy(gb2.at[pl.ds(64, 64)],
                                     sout_hbm.at[pl.ds(HB + r0, 64)],
                                     ssems[1])

            def estore_wait(b):
                r0 = lbase + b * 64
                if last:
                    last_out_dma(r0, False)
                else:
                    pltpu.make_async_copy(gb2.at[pl.ds(0, 64)],
                                          out_hbm.at[pl.ds(HB + r0, 64)],
                                          ssems[0]).wait()
                    pltpu.make_async_copy(gb2.at[pl.ds(64, 64)],
                                          sout_hbm.at[pl.ds(HB + r0, 64)],
                                          ssems[1]).wait()

            eload(0)

            def blk(b, _):
                eload_wait(b)

                @pl.when(b > 0)
                def _():
                    estore_wait(b - 1)

                def grp(g, _):
                    dvec = disv[pl.ds(b * 64 + g * LANES, LANES)]
                    for i in range(LANES):
                        r = g * LANES + i
                        dsc = dvec[i]
                        for q in range(NQ):
                            sl = pl.ds(q * LANES, LANES)
                            a = ga[r, sl] * dsc
                            if last:
                                gb2[r, sl] = (ga[64 + r, sl] + a) * scale
                            else:
                                gb2[r, sl] = ga[64 + r, sl] + a
                                gb2[64 + r, sl] = a * dsc
                    return 0
                lax.fori_loop(0, 64 // LANES, grp, 0)
                estore(b)

                @pl.when(b + 1 < ep_blk)
                def _():
                    eload(b + 1)
                return 0
            lax.fori_loop(0, ep_blk, blk, 0)
            estore_wait(ep_blk - 1)

        return klayer

    consts = dict(H=H, NSLOT=NSLOT, E_PAD=E_PAD, HPAD=HPAD, NCH=NCH,
                  chunk=chunk)
    return k0, make_layer(False), make_layer(True), consts


def kernel(edge_index, users_emb, items_emb):
    k0, klayer, klayer_last, cc = _build(NU, NI, E, 25, 128)
    H, NSLOT, E_PAD = cc["H"], cc["NSLOT"], cc["E_PAD"]
    NCH, chunk = cc["NCH"], cc["chunk"]
    row = edge_index[0].astype(jnp.int32)
    col = edge_index[1].astype(jnp.int32)
    npad = E_PAD - E
    rowp = jnp.concatenate([row, jnp.zeros((npad,), jnp.int32)])
    colp = jnp.concatenate([col, jnp.full((npad,), NSLOT + 1024, jnp.int32)])
    pairs = jnp.stack([rowp.reshape(NS * NCH, chunk),
                       colp.reshape(NS * NCH, chunk)], axis=1).reshape(-1)
    emb0 = (jnp.zeros((NSLOT, D), jnp.float32)
            .at[:NU].set(users_emb)
            .at[H:H + NI].set(items_emb))
    dis, s0, plist, cnt = k0(pairs, emb0)
    m1, s1 = klayer(plist, cnt, s0, dis, emb0)
    m2, s2 = klayer(plist, cnt, s1, dis, m1)
    fin_u, fin_i = klayer_last(plist, cnt, s2, dis, m2)
    return (fin_u, users_emb, fin_i, items_emb)


# speedup vs baseline: 1.1110x; 1.0023x over previous
import functools

import jax
import jax.numpy as jnp
from jax import lax
from jax.experimental import pallas as pl
from jax.experimental.pallas import tpu as pltpu
from jax.experimental.pallas import tpu_sc as plsc

NC = 2
NS = 16
LANES = 16

NU = 25000
NI = 25000
D = 64
E = 800000
LAYERS = 3


def _rsqrt_newton(d):
    i = lax.bitcast_convert_type(d, jnp.int32)
    i = jnp.int32(0x5F3759DF) - lax.shift_right_logical(i, 1)
    y = lax.bitcast_convert_type(i, jnp.float32)
    for _ in range(3):
        y = y * (jnp.float32(1.5) - jnp.float32(0.5) * d * y * y)
    return y


@functools.lru_cache(maxsize=None)
def _build(nu, ni, e, ep_blk, chunk, interpret=False):
    H = NS * 64 * ep_blk
    NSLOT = 2 * H
    assert nu <= H and ni <= H
    RPT = H // NS
    SUP = 3
    NBUF = 3
    align = SUP * chunk
    EPT = -(-e // (NS * align)) * align
    E_PAD = EPT * NS
    NCH = EPT // chunk
    NSUP = NCH // SUP
    CW = 2 * chunk
    SW = SUP * CW
    HPAD = H - nu
    ACCR = H + LANES
    DEGN = -(-(NSLOT + LANES) // (NS * LANES)) * (NS * LANES)
    CAPCH = NCH + 3
    CAPW = CAPCH * CW
    assert RPT % LANES == 0 and chunk % LANES == 0 and NSUP >= 3
    assert EPT % chunk == 0
    NQ = D // LANES
    NG = chunk // LANES

    mesh = plsc.VectorSubcoreMesh(
        core_axis_name="c", subcore_axis_name="s", num_cores=NC,
        num_subcores=NS)
    f32 = jnp.float32
    i32 = jnp.int32
    cparams = pltpu.CompilerParams(
        needs_layout_passes=False, use_tc_tiling_on_sc=False)

    @functools.partial(
        pl.kernel,
        out_type=[jax.ShapeDtypeStruct((NSLOT,), f32),
                  jax.ShapeDtypeStruct((NSLOT, D), f32),
                  jax.ShapeDtypeStruct((NC * NS * CAPW,), i32),
                  jax.ShapeDtypeStruct((NC * NS * LANES,), i32)],
        mesh=mesh,
        scratch_types=[
            pltpu.VMEM((DEGN,), f32),
            pltpu.VMEM((2 * SW,), i32),
            pltpu.VMEM((chunk + 144,), i32),
            pltpu.VMEM((chunk + 144,), i32),
            pltpu.VMEM((CW,), i32),
            pltpu.VMEM((LANES,), i32),
            pltpu.VMEM((RPT,), f32),
            pltpu.VMEM((RPT,), f32),
            pltpu.VMEM((RPT,), f32),
            pltpu.VMEM((64, D), f32),
            pltpu.VMEM((64, D), f32),
            pltpu.VMEM_SHARED((NS * H,), f32),
            pltpu.SemaphoreType.DMA,
            pltpu.SemaphoreType.DMA,
            pltpu.SemaphoreType.DMA,
        ],
        compiler_params=cparams,
        interpret=interpret,
    )
    def k0(pairs_hbm, emb0_hbm, dis_hbm, s_hbm, plist_hbm, cnt_hbm,
           degbuf, stage, prow, pcol, flushbuf, cntbuf, dv, pbuf, disv,
           ebuf, sbuf, degsh, sg0, sg1, fsem):
        c = lax.axis_index("c")
        s = lax.axis_index("s")
        iota = lax.iota(i32, LANES)
        zero16 = jnp.zeros((LANES,), f32)
        one16 = jnp.ones((LANES,), f32)
        tbase = s * NCH * CW
        obase = (c * NS + s) * CAPW
        HB = c * H

        def zdeg(i, _):
            degbuf[pl.ds(i * LANES, LANES)] = zero16
            return 0
        lax.fori_loop(0, DEGN // LANES, zdeg, 0)

        pltpu.async_copy(flushbuf,
                         plist_hbm.at[pl.ds(obase + (CAPCH - 1) * CW, CW)],
                         fsem)

        def flush(p, cnt):
            pltpu.make_async_copy(
                flushbuf, plist_hbm.at[pl.ds(obase, CW)], fsem).wait()
            for i in range(NG):
                sl = pl.ds(i * LANES, LANES)
                flushbuf[sl] = prow[sl]
                flushbuf[pl.ds(chunk + i * LANES, LANES)] = pcol[sl]
            pltpu.async_copy(
                flushbuf, plist_hbm.at[pl.ds(obase + cnt * CW, CW)], fsem)
            for i in range(NG):
                sl = pl.ds(i * LANES, LANES)
                prow[sl] = prow[pl.ds(chunk + i * LANES, LANES)]
                pcol[sl] = pcol[pl.ds(chunk + i * LANES, LANES)]
            return p - chunk

        def do_chunk(u, j, p, cnt):
            sb = (u % 2) * SW + j * CW
            for jj in range(NG):
                cs = stage[pl.ds(sb + chunk + jj * LANES, LANES)]
                slot = cs + jnp.where(cs >= nu, i32(HPAD), i32(0))
                cslot = jnp.where(slot >= NSLOT, NSLOT + iota, slot)
                plsc.addupdate_scatter(degbuf, [cslot], one16)
                r = stage[pl.ds(sb + jj * LANES, LANES)]
                rs = r + jnp.where(r >= nu, i32(HPAD), i32(0))
                loc = slot - HB
                ok = (loc >= 0) & (loc < H)
                plsc.store_compressed(prow.at[pl.ds(p, LANES)], rs, mask=ok)
                plsc.store_compressed(pcol.at[pl.ds(p, LANES)], loc, mask=ok)
                p = p + plsc.all_reduce_population_count(ok)[0]

            @pl.when(p >= chunk)
            def _():
                flush(p, cnt)
            cntn = cnt + jnp.where(p >= chunk, 1, 0)
            pn = jnp.where(p >= chunk, p - chunk, p)
            return pn, cntn

        pltpu.sync_copy(pairs_hbm.at[pl.ds(tbase, SW)], stage.at[pl.ds(0, SW)])
        pltpu.async_copy(pairs_hbm.at[pl.ds(tbase + SW, SW)],
                         stage.at[pl.ds(SW, SW)], sg1)

        def sloop(u, carry):
            p, cnt = carry
            for j in range(SUP):
                p, cnt = do_chunk(u, j, p, cnt)

            @pl.when((u % 2 == 0) & (u < NSUP - 2))
            def _():
                pltpu.async_copy(
                    pairs_hbm.at[pl.ds(tbase + (u + 2) * SW, SW)],
                    stage.at[pl.ds(0, SW)], sg0)

            @pl.when((u % 2 == 1) & (u < NSUP - 2))
            def _():
                pltpu.async_copy(
                    pairs_hbm.at[pl.ds(tbase + (u + 2) * SW, SW)],
                    stage.at[pl.ds(SW, SW)], sg1)
            @pl.when((u + 1) % 2 == 0)
            def _():
                pltpu.make_async_copy(
                    pairs_hbm.at[pl.ds(tbase, SW)],
                    stage.at[pl.ds(0, SW)], sg0).wait()

            @pl.when((u + 1) % 2 == 1)
            def _():
                pltpu.make_async_copy(
                    pairs_hbm.at[pl.ds(tbase, SW)],
                    stage.at[pl.ds(SW, SW)], sg1).wait()
            return p, cnt
        p, cnt = lax.fori_loop(0, NSUP - 1, sloop,
                               (jnp.int32(0), jnp.int32(0)))
        for j in range(SUP):
            p, cnt = do_chunk(NSUP - 1, j, p, cnt)

        @pl.when(p > 0)
        def _():
            for i in range(NG):
                prow[pl.ds(p + i * LANES, LANES)] = jnp.zeros((LANES,), i32)
                pcol[pl.ds(p + i * LANES, LANES)] = H + iota
            flush(jnp.int32(chunk), cnt)
        cnt = jnp.where(p > 0, cnt + 1, cnt)

        def dummy_flush(cnt):
            pltpu.make_async_copy(
                flushbuf, plist_hbm.at[pl.ds(obase, CW)], fsem).wait()
            for i in range(NG):
                flushbuf[pl.ds(i * LANES, LANES)] = jnp.zeros((LANES,), i32)
                flushbuf[pl.ds(chunk + i * LANES, LANES)] = H + iota
            pltpu.async_copy(
                flushbuf, plist_hbm.at[pl.ds(obase + cnt * CW, CW)], fsem)

        for _ in range(SUP - 1):
            @pl.when(cnt % SUP != 0)
            def _():
                dummy_flush(cnt)
            cnt = jnp.where(cnt % SUP != 0, cnt + 1, cnt)
        pltpu.make_async_copy(
            flushbuf, plist_hbm.at[pl.ds(obase, CW)], fsem).wait()
        cntbuf[pl.ds(0, LANES)] = jnp.zeros((LANES,), i32) + cnt
        pltpu.sync_copy(cntbuf, cnt_hbm.at[pl.ds((c * NS + s) * LANES, LANES)])

        pltpu.sync_copy(degbuf.at[pl.ds(c * H, H)], degsh.at[pl.ds(s * H, H)])
        plsc.subcore_barrier()

        gbase = c * H + s * RPT

        def zdv(i, _):
            dv[pl.ds(i * LANES, LANES)] = zero16
            return 0
        lax.fori_loop(0, RPT // LANES, zdv, 0)
        for t in range(NS):
            pltpu.sync_copy(degsh.at[pl.ds(t * H + s * RPT, RPT)], pbuf)

            def acc_part(i, _):
                sl = pl.ds(i * LANES, LANES)
                dv[sl] = dv[sl] + pbuf[sl]
                return 0
            lax.fori_loop(0, RPT // LANES, acc_part, 0)

        def nr(i, _):
            d = dv[pl.ds(i * LANES, LANES)]
            y = _rsqrt_newton(d)
            disv[pl.ds(i * LANES, LANES)] = jnp.where(
                d > jnp.float32(0.5), y, jnp.float32(0.0))
            return 0
        lax.fori_loop(0, RPT // LANES, nr, 0)
        pltpu.sync_copy(disv, dis_hbm.at[pl.ds(gbase, RPT)])

        def eb_load(b):
            pltpu.async_copy(emb0_hbm.at[pl.ds(gbase + b * 64, 64)], ebuf,
                             sg0)

        def eb_wait(b):
            pltpu.make_async_copy(emb0_hbm.at[pl.ds(gbase + b * 64, 64)],
                                  ebuf, sg0).wait()

        def sb_store(b):
            pltpu.async_copy(sbuf, s_hbm.at[pl.ds(gbase + b * 64, 64)], sg1)

        def sb_wait(b):
            pltpu.make_async_copy(sbuf, s_hbm.at[pl.ds(gbase + b * 64, 64)],
                                  sg1).wait()

        eb_load(0)

        def blk(b, _):
            eb_wait(b)

            @pl.when(b > 0)
            def _():
                sb_wait(b - 1)

            def grp(g, _):
                dvec = disv[pl.ds(b * 64 + g * LANES, LANES)]
                for i in range(LANES):
                    r = g * LANES + i
                    dsc = dvec[i]
                    for q in range(NQ):
                        sl = pl.ds(q * LANES, LANES)
                        sbuf[r, sl] = ebuf[r, sl] * dsc
                return 0
            lax.fori_loop(0, 64 // LANES, grp, 0)
            sb_store(b)

            @pl.when(b + 1 < ep_blk)
            def _():
                eb_load(b + 1)
            return 0
        lax.fori_loop(0, ep_blk, blk, 0)
        sb_wait(ep_blk - 1)

    def make_layer(last):
        if last:
            outs = [jax.ShapeDtypeStruct((nu, D), f32),
                    jax.ShapeDtypeStruct((ni, D), f32)]
        else:
            outs = [jax.ShapeDtypeStruct((NSLOT, D), f32),
                    jax.ShapeDtypeStruct((NSLOT, D), f32)]

        @functools.partial(
            pl.kernel,
            out_type=outs,
            mesh=mesh,
            scratch_types=(
                [pltpu.VMEM((chunk,), i32) for _ in range(NBUF)]
                + [pltpu.VMEM((chunk,), i32) for _ in range(NBUF)]
                + [pltpu.VMEM((chunk, D), f32) for _ in range(NBUF)]
                + [
                    pltpu.VMEM((2 * SW,), i32),
                    pltpu.VMEM((LANES,), i32),
                    pltpu.VMEM((RPT,), f32),
                    pltpu.VMEM_SHARED((ACCR, D), f32),
                ]
                + [pltpu.SemaphoreType.DMA for _ in range(2 * NBUF + 2)]
            ),
            compiler_params=cparams,
            interpret=interpret,
        )
        def klayer(plist_hbm, cnt_hbm, s_hbm, dis_hbm, msum_hbm, *rest):
            if last:
                uout_hbm, iout_hbm, *rest = rest
                out_hbm = sout_hbm = None
            else:
                out_hbm, sout_hbm, *rest = rest
                uout_hbm = iout_hbm = None
            rowbs = rest[0:NBUF]
            colbs = rest[NBUF:2 * NBUF]
            gbufs = rest[2 * NBUF:3 * NBUF]
            (stage, cntb, disv, accsh) = rest[3 * NBUF:3 * NBUF + 4]
            gsems = rest[3 * NBUF + 4:3 * NBUF + 4 + NBUF]
            ssems = rest[3 * NBUF + 4 + NBUF:3 * NBUF + 4 + 2 * NBUF]
            sg0, sg1 = rest[3 * NBUF + 4 + 2 * NBUF:]
            c = lax.axis_index("c")
            s = lax.axis_index("s")
            zero16 = jnp.zeros((LANES,), f32)
            HB = c * H
            tbase = (c * NS + s) * CAPW

            pltpu.sync_copy(cnt_hbm.at[pl.ds((c * NS + s) * LANES, LANES)],
                            cntb)
            nch = cntb[pl.ds(0, LANES)][0]
            nsup = nch // SUP

            def zrow(i, _):
                for q in range(NQ):
                    gbufs[0][i, pl.ds(q * LANES, LANES)] = zero16
                return 0
            lax.fori_loop(0, 64, zrow, 0)

            def zb(b, _):
                idx = b * NS + s
                pltpu.sync_copy(gbufs[0].at[pl.ds(0, 64)],
                                accsh.at[pl.ds(idx * 64, 64)])
                return 0
            lax.fori_loop(0, H // (64 * NS), zb, 0)
            plsc.subcore_barrier()

            def fire(u, j):
                sb = (u % 2) * SW + j * CW
                for jj in range(NG):
                    sl = pl.ds(jj * LANES, LANES)
                    rowbs[j][sl] = stage[pl.ds(sb + jj * LANES, LANES)]
                    colbs[j][sl] = stage[pl.ds(sb + chunk + jj * LANES,
                                               LANES)]
                pltpu.async_copy(s_hbm.at[rowbs[j]], gbufs[j], gsems[j])

            def scat(j):
                pltpu.make_async_copy(
                    s_hbm.at[rowbs[j]], gbufs[j], gsems[j]).wait()
                pltpu.async_copy(
                    gbufs[j], accsh.at[colbs[j]], ssems[j], add=True)

            def scat_wait(j):
                pltpu.make_async_copy(
                    gbufs[j], accsh.at[colbs[j]], ssems[j]).wait()

            @pl.when(nsup > 0)
            def _():
                pltpu.sync_copy(plist_hbm.at[pl.ds(tbase, SW)],
                                stage.at[pl.ds(0, SW)])

                @pl.when(nsup >= 2)
                def _():
                    pltpu.async_copy(plist_hbm.at[pl.ds(tbase + SW, SW)],
                                     stage.at[pl.ds(SW, SW)], sg1)
                for j in range(SUP):
                    fire(0, j)

                def outer(u, _):
                    @pl.when((u % 2 == 0) & (u < nsup - 2))
                    def _():
                        pltpu.async_copy(
                            plist_hbm.at[pl.ds(tbase + (u + 2) * SW, SW)],
                            stage.at[pl.ds(0, SW)], sg0)

                    @pl.when((u % 2 == 1) & (u < nsup - 2))
                    def _():
                        pltpu.async_copy(
                            plist_hbm.at[pl.ds(tbase + (u + 2) * SW, SW)],
                            stage.at[pl.ds(SW, SW)], sg1)
                    @pl.when((u + 1) % 2 == 0)
                    def _():
                        pltpu.make_async_copy(
                            plist_hbm.at[pl.ds(tbase, SW)],
                            stage.at[pl.ds(0, SW)], sg0).wait()

                    @pl.when((u + 1) % 2 == 1)
                    def _():
                        pltpu.make_async_copy(
                            plist_hbm.at[pl.ds(tbase, SW)],
                            stage.at[pl.ds(SW, SW)], sg1).wait()
                    for j in range(SUP):
                        scat(j)
                        scat_wait(j)
                        fire(u + 1, j)
                    return 0
                lax.fori_loop(0, nsup - 1, outer, 0)
                for j in range(SUP):
                    scat(j)
                for j in range(SUP):
                    scat_wait(j)
            plsc.subcore_barrier()

            lbase = s * RPT
            scale = jnp.float32(1.0 / (LAYERS + 1))
            ga, gb2 = gbufs[0], gbufs[1]
            pltpu.sync_copy(dis_hbm.at[pl.ds(HB + lbase, RPT)], disv)

            def eload(b):
                r0 = lbase + b * 64
                pltpu.async_copy(accsh.at[pl.ds(r0, 64)],
                                 ga.at[pl.ds(0, 64)], gsems[0])
                pltpu.async_copy(msum_hbm.at[pl.ds(HB + r0, 64)],
                                 ga.at[pl.ds(64, 64)], gsems[1])

            def eload_wait(b):
                r0 = lbase + b * 64
                pltpu.make_async_copy(accsh.at[pl.ds(r0, 64)],
                                      ga.at[pl.ds(0, 64)], gsems[0]).wait()
                pltpu.make_async_copy(msum_hbm.at[pl.ds(HB + r0, 64)],
                                      ga.at[pl.ds(64, 64)], gsems[1]).wait()

            NR = nu % 64

            def last_out_dma(r0, issue):
                for half, dst in ((0, uout_hbm), (1, iout_hbm)):
                    nreal = nu if half == 0 else ni

                    @pl.when((c == half) & (r0 + 64 <= nreal))
                    def _():
                        cp = pltpu.make_async_copy(
                            gb2.at[pl.ds(0, 64)], dst.at[pl.ds(r0, 64)],
                            ssems[0])
                        cp.start() if issue else cp.wait()
                    if NR > 0:
                        @pl.when((c == half) & (r0 < nreal)
                                 & (r0 + 64 > nreal))
                        def _():
                            cp = pltpu.make_async_copy(
                                gb2.at[pl.ds(0, NR)],
                                dst.at[pl.ds(r0, NR)], ssems[0])
                            cp.start() if issue else cp.wait()

            def estore(b):
                r0 = lbase + b * 64
                if last:
                    last_out_dma(r0, True)
                else:
                    pltpu.async_copy(gb2.at[pl.ds(0, 64)],
                                     out_hbm.at[pl.ds(HB + r0, 64)], ssems[0])
                    pltpu.async_copy(gb2.at[pl.ds(64, 64)],
                                     sout_hbm.at[pl.ds(HB + r0, 64)],
                                     ssems[1])

            def estore_wait(b):
                r0 = lbase + b * 64
                if last:
                    last_out_dma(r0, False)
                else:
                    pltpu.make_async_copy(gb2.at[pl.ds(0, 64)],
                                          out_hbm.at[pl.ds(HB + r0, 64)],
                                          ssems[0]).wait()
                    pltpu.make_async_copy(gb2.at[pl.ds(64, 64)],
                                          sout_hbm.at[pl.ds(HB + r0, 64)],
                                          ssems[1]).wait()

            eload(0)

            def blk(b, _):
                eload_wait(b)

                @pl.when(b > 0)
                def _():
                    estore_wait(b - 1)

                def grp(g, _):
                    dvec = disv[pl.ds(b * 64 + g * LANES, LANES)]
                    for i in range(LANES):
                        r = g * LANES + i
                        dsc = dvec[i]
                        for q in range(NQ):
                            sl = pl.ds(q * LANES, LANES)
                            a = ga[r, sl] * dsc
                            if last:
                                gb2[r, sl] = (ga[64 + r, sl] + a) * scale
                            else:
                                gb2[r, sl] = ga[64 + r, sl] + a
                                gb2[64 + r, sl] = a * dsc
                    return 0
                lax.fori_loop(0, 64 // LANES, grp, 0)
                estore(b)

                @pl.when(b + 1 < ep_blk)
                def _():
                    eload(b + 1)
                return 0
            lax.fori_loop(0, ep_blk, blk, 0)
            estore_wait(ep_blk - 1)

        return klayer

    consts = dict(H=H, NSLOT=NSLOT, E_PAD=E_PAD, HPAD=HPAD, NCH=NCH,
                  chunk=chunk)
    return k0, make_layer(False), make_layer(True), consts


def kernel(edge_index, users_emb, items_emb):
    k0, klayer, klayer_last, cc = _build(NU, NI, E, 25, 128)
    H, NSLOT, E_PAD = cc["H"], cc["NSLOT"], cc["E_PAD"]
    NCH, chunk = cc["NCH"], cc["chunk"]
    row = edge_index[0].astype(jnp.int32)
    col = edge_index[1].astype(jnp.int32)
    npad = E_PAD - E
    rowp = jnp.concatenate([row, jnp.zeros((npad,), jnp.int32)])
    colp = jnp.concatenate([col, jnp.full((npad,), NSLOT + 1024, jnp.int32)])
    pairs = jnp.stack([rowp.reshape(NS * NCH, chunk),
                       colp.reshape(NS * NCH, chunk)], axis=1).reshape(-1)
    emb0 = (jnp.zeros((NSLOT, D), jnp.float32)
            .at[:NU].set(users_emb)
            .at[H:H + NI].set(items_emb))
    dis, s0, plist, cnt = k0(pairs, emb0)
    m1, s1 = klayer(plist, cnt, s0, dis, emb0)
    m2, s2 = klayer(plist, cnt, s1, dis, m1)
    fin_u, fin_i = klayer_last(plist, cnt, s2, dis, m2)
    return (fin_u, users_emb, fin_i, items_emb)
